# Initial kernel scaffold; baseline (speedup 1.0000x reference)
import jax
import jax.numpy as jnp
from jax.experimental import pallas as pl


def kernel(feature, edge_index, batch, emb_W, emb_b, A_W, A_b, B_W, B_b, D_W, D_b, E_W, E_b, bn_gamma, bn_beta, W1, b1, W2, b2, W3, b3):
    raise NotImplementedError("write your pallas kernel here")


# speedup vs baseline: 1.3548x; 1.3548x over previous
import functools

import jax
import jax.numpy as jnp
from jax import lax
from jax.experimental import pallas as pl
from jax.experimental.pallas import tpu as pltpu
from jax.experimental.pallas import tpu_sc as plsc

N = 10000
E = 320000
H = 128
L = 4
G = 64
NCLS = 10

NSC = 2
NTEC = 16
CH = 128
EPT = 20480
E2 = EPT * NTEC
NCHUNK = EPT // CH
RPT = 624

BLK = 1000
NB = N // BLK


def _embed_body(x_ref, w_ref, b_ref, o_ref):
    o_ref[...] = (
        jnp.dot(x_ref[...], w_ref[...], preferred_element_type=jnp.float32)
        + b_ref[...]
    )


def _embed(feature, emb_W, emb_b):
    return pl.pallas_call(
        _embed_body,
        grid=(NB,),
        in_specs=[
            pl.BlockSpec((BLK, H), lambda i: (i, 0)),
            pl.BlockSpec((H, H), lambda i: (0, 0)),
            pl.BlockSpec((1, H), lambda i: (0, 0)),
        ],
        out_specs=pl.BlockSpec((BLK, H), lambda i: (i, 0)),
        out_shape=jax.ShapeDtypeStruct((N, H), jnp.float32),
    )(feature, emb_W, emb_b.reshape(1, H))


def _mm_body(h_ref, w_ref, b_ref, a_ref, d_ref, eb_ref):
    out = (
        jnp.dot(h_ref[...], w_ref[0], preferred_element_type=jnp.float32)
        + b_ref[0]
    )
    d_ref[...] = out[:, 0:128]
    a_ref[...] = out[:, 128:192]
    eb_ref[...] = out[:, 192:320]


def _mm(h, Wc, bc):
    return pl.pallas_call(
        _mm_body,
        grid=(NB, 2),
        in_specs=[
            pl.BlockSpec((BLK, H), lambda i, c: (i, 0)),
            pl.BlockSpec((1, H, 5 * 64), lambda i, c: (c, 0, 0)),
            pl.BlockSpec((1, 1, 5 * 64), lambda i, c: (c, 0, 0)),
        ],
        out_specs=[
            pl.BlockSpec((BLK, 64), lambda i, c: (c * NB + i, 0)),
            pl.BlockSpec((BLK, H), lambda i, c: (i, 0)),
            pl.BlockSpec((BLK, H), lambda i, c: (c * NB + i, 0)),
        ],
        out_shape=[
            jax.ShapeDtypeStruct((2 * N, 64), jnp.float32),
            jax.ShapeDtypeStruct((N, H), jnp.float32),
            jax.ShapeDtypeStruct((2 * N, H), jnp.float32),
        ],
    )(h, Wc, bc)


def _norm_mm_body(hp_ref, st_ref, g_ref, bt_ref, hin_ref, w_ref, b_ref,
                  h_ref, a_ref, d_ref, eb_ref):
    mean = st_ref[0] * (1.0 / N)
    var = st_ref[1] * (1.0 / N) - mean * mean
    rstd = lax.rsqrt(var + 1e-5)
    h = (
        jnp.maximum((hp_ref[...] - mean) * (rstd * g_ref[0]) + bt_ref[0], 0.0)
        + hin_ref[...]
    )
    h_ref[...] = h
    out = jnp.dot(h, w_ref[0], preferred_element_type=jnp.float32) + b_ref[0]
    d_ref[...] = out[:, 0:128]
    a_ref[...] = out[:, 128:192]
    eb_ref[...] = out[:, 192:320]


def _norm_mm(hpre, stats, gamma, beta, h_in, Wc, bc):
    return pl.pallas_call(
        _norm_mm_body,
        grid=(NB, 2),
        in_specs=[
            pl.BlockSpec((BLK, H), lambda i, c: (i, 0)),
            pl.BlockSpec((2, H), lambda i, c: (0, 0)),
            pl.BlockSpec((1, H), lambda i, c: (0, 0)),
            pl.BlockSpec((1, H), lambda i, c: (0, 0)),
            pl.BlockSpec((BLK, H), lambda i, c: (i, 0)),
            pl.BlockSpec((1, H, 5 * 64), lambda i, c: (c, 0, 0)),
            pl.BlockSpec((1, 1, 5 * 64), lambda i, c: (c, 0, 0)),
        ],
        out_specs=[
            pl.BlockSpec((BLK, H), lambda i, c: (i, 0)),
            pl.BlockSpec((BLK, 64), lambda i, c: (c * NB + i, 0)),
            pl.BlockSpec((BLK, H), lambda i, c: (i, 0)),
            pl.BlockSpec((BLK, H), lambda i, c: (c * NB + i, 0)),
        ],
        out_shape=[
            jax.ShapeDtypeStruct((N, H), jnp.float32),
            jax.ShapeDtypeStruct((2 * N, 64), jnp.float32),
            jax.ShapeDtypeStruct((N, H), jnp.float32),
            jax.ShapeDtypeStruct((2 * N, H), jnp.float32),
        ],
    )(hpre, stats, gamma.reshape(1, H), beta.reshape(1, H), h_in, Wc, bc)


def _update_body(a0_ref, a1_ref, nd0_ref, nd1_ref, hp_ref, st_ref, acc_ref):
    i = pl.program_id(0)
    nd0 = nd0_ref[...]
    nd1 = nd1_ref[...]
    num = jnp.concatenate([nd0[:, 64:128], nd1[:, 64:128]], axis=1)
    den = jnp.concatenate([nd0[:, 0:64], nd1[:, 0:64]], axis=1)
    ah = jnp.concatenate([a0_ref[...], a1_ref[...]], axis=1)
    hp = ah + num / (den + 1e-6)
    hp_ref[...] = hp
    s1 = jnp.sum(hp, axis=0, keepdims=True)
    s2 = jnp.sum(hp * hp, axis=0, keepdims=True)
    s = jnp.concatenate([s1, s2], axis=0)

    @pl.when(i == 0)
    def _():
        acc_ref[...] = s

    @pl.when(i > 0)
    def _():
        acc_ref[...] += s

    st_ref[...] = acc_ref[...]


def _update(Ah2, numden):
    return pl.pallas_call(
        _update_body,
        grid=(NB,),
        in_specs=[
            pl.BlockSpec((BLK, 64), lambda i: (i, 0)),
            pl.BlockSpec((BLK, 64), lambda i: (NB + i, 0)),
            pl.BlockSpec((BLK, H), lambda i: (i, 0)),
            pl.BlockSpec((BLK, H), lambda i: (NB + i, 0)),
        ],
        out_specs=[
            pl.BlockSpec((BLK, H), lambda i: (i, 0)),
            pl.BlockSpec((2, H), lambda i: (0, 0)),
        ],
        out_shape=[
            jax.ShapeDtypeStruct((N, H), jnp.float32),
            jax.ShapeDtypeStruct((2, H), jnp.float32),
        ],
        scratch_shapes=[pltpu.VMEM((2, H), jnp.float32)],
    )(Ah2, Ah2, numden, numden)


def _read_body(hp_ref, st_ref, g_ref, bt_ref, hin_ref, b_ref,
               w1_ref, b1_ref, w2_ref, b2_ref, w3_ref, b3_ref,
               y_ref, hs_ref, cnt_ref):
    i = pl.program_id(0)
    mean = st_ref[0] * (1.0 / N)
    var = st_ref[1] * (1.0 / N) - mean * mean
    rstd = lax.rsqrt(var + 1e-5)
    h = (
        jnp.maximum((hp_ref[...] - mean) * (rstd * g_ref[0]) + bt_ref[0], 0.0)
        + hin_ref[...]
    )
    gid = lax.broadcasted_iota(jnp.int32, (BLK, G), 1)
    oh = (b_ref[...] == gid).astype(jnp.float32)
    dnums = (((0,), (0,)), ((), ()))
    hs = lax.dot_general(oh, h, dnums, preferred_element_type=jnp.float32)
    cn = lax.dot_general(oh, jnp.ones_like(h), dnums,
                         preferred_element_type=jnp.float32)

    @pl.when(i == 0)
    def _():
        hs_ref[...] = hs
        cnt_ref[...] = cn

    @pl.when(i > 0)
    def _():
        hs_ref[...] += hs
        cnt_ref[...] += cn

    @pl.when(i == NB - 1)
    def _():
        hg = hs_ref[...] / jnp.maximum(cnt_ref[...], 1.0)
        y1 = jnp.maximum(
            jnp.dot(hg, w1_ref[...], preferred_element_type=jnp.float32)
            + b1_ref[0], 0.0)
        y2 = jnp.maximum(
            jnp.dot(y1, w2_ref[...], preferred_element_type=jnp.float32)
            + b2_ref[0], 0.0)
        y_ref[...] = (
            jnp.dot(y2, w3_ref[...], preferred_element_type=jnp.float32)
            + b3_ref[0]
        )


def _readout(hpre, stats, gamma, beta, h_in, batch2d, W1, b1, W2, b2, W3p, b3p):
    return pl.pallas_call(
        _read_body,
        grid=(NB,),
        in_specs=[
            pl.BlockSpec((BLK, H), lambda i: (i, 0)),
            pl.BlockSpec((2, H), lambda i: (0, 0)),
            pl.BlockSpec((1, H), lambda i: (0, 0)),
            pl.BlockSpec((1, H), lambda i: (0, 0)),
            pl.BlockSpec((BLK, H), lambda i: (i, 0)),
            pl.BlockSpec((BLK, 1), lambda i: (i, 0)),
            pl.BlockSpec((H, 64), lambda i: (0, 0)),
            pl.BlockSpec((1, 64), lambda i: (0, 0)),
            pl.BlockSpec((64, 32), lambda i: (0, 0)),
            pl.BlockSpec((1, 32), lambda i: (0, 0)),
            pl.BlockSpec((32, H), lambda i: (0, 0)),
            pl.BlockSpec((1, H), lambda i: (0, 0)),
        ],
        out_specs=pl.BlockSpec((G, H), lambda i: (0, 0)),
        out_shape=jax.ShapeDtypeStruct((G, H), jnp.float32),
        scratch_shapes=[
            pltpu.VMEM((G, H), jnp.float32),
            pltpu.VMEM((G, H), jnp.float32),
        ],
    )(hpre, stats, gamma.reshape(1, H), beta.reshape(1, H), h_in, batch2d,
      W1, b1.reshape(1, 64), W2, b2.reshape(1, 32), W3p, b3p.reshape(1, H))


def _edge_body(dh_hbm, eb2_hbm, dsts_hbm, idxd_hbm, idxeb_hbm, out_hbm,
               dst_v, idd_v, ideb_v, drows, ebrows, acc, sem1, sem2):
    c = lax.axis_index("c")
    s = lax.axis_index("s")

    def _zb16(j, carry):
        r = j // 8
        q = j % 8
        ebrows[r, pl.ds(q * 16, 16)] = jnp.zeros((16,), jnp.float32)
        return carry

    lax.fori_loop(0, CH * 8, _zb16, 0)
    row0 = s * RPT
    for t in range(4):
        pltpu.sync_copy(ebrows, acc.at[pl.ds(row0 + t * CH, CH)])
    pltpu.sync_copy(ebrows.at[pl.ds(0, RPT - 4 * CH)],
                    acc.at[pl.ds(row0 + 4 * CH, RPT - 4 * CH)])

    @pl.when(s == NTEC - 1)
    def _():
        pltpu.sync_copy(ebrows.at[pl.ds(0, 24)],
                        acc.at[pl.ds(NTEC * RPT, 24)])

    plsc.subcore_barrier()

    base0 = s * EPT

    def _chunk(k, carry):
        base = base0 + k * CH
        pltpu.sync_copy(dsts_hbm.at[pl.ds(base, CH)], dst_v)
        pltpu.sync_copy(idxd_hbm.at[pl.ds(base, CH)], idd_v)
        pltpu.sync_copy(idxeb_hbm.at[c, pl.ds(base, CH)], ideb_v)
        cp1 = pltpu.async_copy(dh_hbm.at[idd_v], drows, sem1)
        cp2 = pltpu.async_copy(eb2_hbm.at[ideb_v], ebrows, sem2)
        cp1.wait()
        cp2.wait()

        coff = c * 64

        def _row(r, rc):
            for q in range(4):
                d = drows[r, pl.ds(coff + q * 16, 16)]
                e = ebrows[r, pl.ds(q * 16, 16)]
                sig = 1.0 / (1.0 + jnp.exp(-(d + e)))
                ebrows[r, pl.ds(q * 16, 16)] = sig
                b = ebrows[r, pl.ds(64 + q * 16, 16)]
                ebrows[r, pl.ds(64 + q * 16, 16)] = sig * b
            return rc

        lax.fori_loop(0, CH, _row, 0)
        pltpu.sync_copy(ebrows, acc.at[dst_v], add=True)
        return carry

    lax.fori_loop(0, NCHUNK, _chunk, 0)
    plsc.subcore_barrier()
    pltpu.sync_copy(acc.at[pl.ds(row0, RPT)],
                    out_hbm.at[pl.ds(c * N + row0, RPT)])

    @pl.when(s == NTEC - 1)
    def _():
        pltpu.sync_copy(acc.at[pl.ds(NTEC * RPT, 16)],
                        out_hbm.at[pl.ds(c * N + NTEC * RPT, 16)])


def _edge_pass(Dh, EB2, dst_s, idxD, idxEB):
    mesh = plsc.VectorSubcoreMesh(core_axis_name="c", subcore_axis_name="s",
                                  num_cores=NSC, num_subcores=NTEC)
    f = pl.kernel(
        _edge_body,
        out_type=jax.ShapeDtypeStruct((2 * N, H), jnp.float32),
        mesh=mesh,
        scratch_types=[
            pltpu.VMEM((CH,), jnp.int32),
            pltpu.VMEM((CH,), jnp.int32),
            pltpu.VMEM((CH,), jnp.int32),
            pltpu.VMEM((CH, H), jnp.float32),
            pltpu.VMEM((CH, H), jnp.float32),
            pltpu.VMEM_SHARED((N + 8, H), jnp.float32),
            pltpu.SemaphoreType.DMA,
            pltpu.SemaphoreType.DMA,
        ],
    )
    return f(Dh, EB2, dst_s, idxD, idxEB)


def _split_cols(W):
    return jnp.stack([W[:, :, 0:64], W[:, :, 64:128]], axis=1)


def _split_cols_b(b):
    return jnp.stack([b[:, None, 0:64], b[:, None, 64:128]], axis=1)


@jax.jit
def kernel(feature, edge_index, batch, emb_W, emb_b, A_W, A_b, B_W, B_b,
           D_W, D_b, E_W, E_b, bn_gamma, bn_beta, W1, b1, W2, b2, W3, b3):
    src = edge_index[0]
    dst = edge_index[1]
    pad = E2 - E
    zpad = jnp.zeros((pad,), jnp.int32)
    src_g = jnp.concatenate([src, zpad])
    dst_g = jnp.concatenate([dst, zpad])
    dst_s = jnp.concatenate([dst, jnp.full((pad,), N, jnp.int32)])
    idxD = dst_g
    idxEB = jnp.stack([src_g, src_g + N])

    Dfull = jnp.broadcast_to(D_W[:, None], (L, 2, H, H))
    Dfull_b = jnp.broadcast_to(D_b[:, None, None], (L, 2, 1, H))
    Wc = jnp.concatenate(
        [Dfull, _split_cols(A_W), _split_cols(E_W), _split_cols(B_W)],
        axis=-1)
    bcat = jnp.concatenate(
        [Dfull_b, _split_cols_b(A_b), _split_cols_b(E_b), _split_cols_b(B_b)],
        axis=-1)

    W3p = jnp.pad(W3, ((0, 0), (0, H - NCLS)))
    b3p = jnp.pad(b3, (0, H - NCLS))
    batch2d = batch.reshape(N, 1)

    h = _embed(feature, emb_W, emb_b)
    h_in = h
    Ah2, Dh2, EB2 = _mm(h, Wc[0], bcat[0])
    for l in range(L):
        numden = _edge_pass(Dh2, EB2, dst_s, idxD, idxEB)
        hpre, stats = _update(Ah2, numden)
        if l < L - 1:
            h_in, Ah2, Dh2, EB2 = _norm_mm(
                hpre, stats, bn_gamma[l], bn_beta[l], h_in,
                Wc[l + 1], bcat[l + 1])
        else:
            y = _readout(hpre, stats, bn_gamma[l], bn_beta[l], h_in,
                         batch2d, W1, b1, W2, b2, W3p, b3p)
    return y[:, :NCLS]


# speedup vs baseline: 2.1522x; 1.5885x over previous
import functools

import jax
import jax.numpy as jnp
from jax import lax
from jax.experimental import pallas as pl
from jax.experimental.pallas import tpu as pltpu
from jax.experimental.pallas import tpu_sc as plsc

N = 10000
E = 320000
H = 128
L = 4
G = 64
NCLS = 10

NSC = 2
NTEC = 16
CH = 64
NCHUNK = 314
EPT = NCHUNK * CH
E2 = EPT * NTEC
NPAIR = NCHUNK // 2
RPT = 624

BLK = 1000
NB = N // BLK


def _embed_body(x_ref, w_ref, b_ref, o_ref):
    o_ref[...] = (
        jnp.dot(x_ref[...], w_ref[...], preferred_element_type=jnp.float32)
        + b_ref[...]
    )


def _embed(feature, emb_W, emb_b):
    return pl.pallas_call(
        _embed_body,
        grid=(NB,),
        in_specs=[
            pl.BlockSpec((BLK, H), lambda i: (i, 0)),
            pl.BlockSpec((H, H), lambda i: (0, 0)),
            pl.BlockSpec((1, H), lambda i: (0, 0)),
        ],
        out_specs=pl.BlockSpec((BLK, H), lambda i: (i, 0)),
        out_shape=jax.ShapeDtypeStruct((N, H), jnp.float32),
    )(feature, emb_W, emb_b.reshape(1, H))


def _mm_body(h_ref, w_ref, b_ref, a_ref, d_ref, eb_ref):
    out = (
        jnp.dot(h_ref[...], w_ref[0], preferred_element_type=jnp.float32)
        + b_ref[0]
    )
    d_ref[...] = out[:, 0:128]
    a_ref[...] = out[:, 128:192]
    eb_ref[...] = out[:, 192:320]


def _mm(h, Wc, bc):
    return pl.pallas_call(
        _mm_body,
        grid=(NB, 2),
        in_specs=[
            pl.BlockSpec((BLK, H), lambda i, c: (i, 0)),
            pl.BlockSpec((1, H, 5 * 64), lambda i, c: (c, 0, 0)),
            pl.BlockSpec((1, 1, 5 * 64), lambda i, c: (c, 0, 0)),
        ],
        out_specs=[
            pl.BlockSpec((BLK, 64), lambda i, c: (c * NB + i, 0)),
            pl.BlockSpec((BLK, H), lambda i, c: (i, 0)),
            pl.BlockSpec((BLK, H), lambda i, c: (c * NB + i, 0)),
        ],
        out_shape=[
            jax.ShapeDtypeStruct((2 * N, 64), jnp.float32),
            jax.ShapeDtypeStruct((N, H), jnp.float32),
            jax.ShapeDtypeStruct((2 * N, H), jnp.float32),
        ],
    )(h, Wc, bc)


def _norm_mm_body(hp_ref, st_ref, g_ref, bt_ref, hin_ref, w_ref, b_ref,
                  h_ref, a_ref, d_ref, eb_ref):
    mean = st_ref[0] * (1.0 / N)
    var = st_ref[1] * (1.0 / N) - mean * mean
    rstd = lax.rsqrt(var + 1e-5)
    h = (
        jnp.maximum((hp_ref[...] - mean) * (rstd * g_ref[0]) + bt_ref[0], 0.0)
        + hin_ref[...]
    )
    h_ref[...] = h
    out = jnp.dot(h, w_ref[0], preferred_element_type=jnp.float32) + b_ref[0]
    d_ref[...] = out[:, 0:128]
    a_ref[...] = out[:, 128:192]
    eb_ref[...] = out[:, 192:320]


def _norm_mm(hpre, stats, gamma, beta, h_in, Wc, bc):
    return pl.pallas_call(
        _norm_mm_body,
        grid=(NB, 2),
        in_specs=[
            pl.BlockSpec((BLK, H), lambda i, c: (i, 0)),
            pl.BlockSpec((2, H), lambda i, c: (0, 0)),
            pl.BlockSpec((1, H), lambda i, c: (0, 0)),
            pl.BlockSpec((1, H), lambda i, c: (0, 0)),
            pl.BlockSpec((BLK, H), lambda i, c: (i, 0)),
            pl.BlockSpec((1, H, 5 * 64), lambda i, c: (c, 0, 0)),
            pl.BlockSpec((1, 1, 5 * 64), lambda i, c: (c, 0, 0)),
        ],
        out_specs=[
            pl.BlockSpec((BLK, H), lambda i, c: (i, 0)),
            pl.BlockSpec((BLK, 64), lambda i, c: (c * NB + i, 0)),
            pl.BlockSpec((BLK, H), lambda i, c: (i, 0)),
            pl.BlockSpec((BLK, H), lambda i, c: (c * NB + i, 0)),
        ],
        out_shape=[
            jax.ShapeDtypeStruct((N, H), jnp.float32),
            jax.ShapeDtypeStruct((2 * N, 64), jnp.float32),
            jax.ShapeDtypeStruct((N, H), jnp.float32),
            jax.ShapeDtypeStruct((2 * N, H), jnp.float32),
        ],
    )(hpre, stats, gamma.reshape(1, H), beta.reshape(1, H), h_in, Wc, bc)


def _update_body(a0_ref, a1_ref, nd0_ref, nd1_ref, hp_ref, st_ref, acc_ref):
    i = pl.program_id(0)
    nd0 = nd0_ref[...]
    nd1 = nd1_ref[...]
    num = jnp.concatenate([nd0[:, 64:128], nd1[:, 64:128]], axis=1)
    den = jnp.concatenate([nd0[:, 0:64], nd1[:, 0:64]], axis=1)
    ah = jnp.concatenate([a0_ref[...], a1_ref[...]], axis=1)
    hp = ah + num / (den + 1e-6)
    hp_ref[...] = hp
    s1 = jnp.sum(hp, axis=0, keepdims=True)
    s2 = jnp.sum(hp * hp, axis=0, keepdims=True)
    s = jnp.concatenate([s1, s2], axis=0)

    @pl.when(i == 0)
    def _():
        acc_ref[...] = s

    @pl.when(i > 0)
    def _():
        acc_ref[...] += s

    st_ref[...] = acc_ref[...]


def _update(Ah2, numden):
    return pl.pallas_call(
        _update_body,
        grid=(NB,),
        in_specs=[
            pl.BlockSpec((BLK, 64), lambda i: (i, 0)),
            pl.BlockSpec((BLK, 64), lambda i: (NB + i, 0)),
            pl.BlockSpec((BLK, H), lambda i: (i, 0)),
            pl.BlockSpec((BLK, H), lambda i: (NB + i, 0)),
        ],
        out_specs=[
            pl.BlockSpec((BLK, H), lambda i: (i, 0)),
            pl.BlockSpec((2, H), lambda i: (0, 0)),
        ],
        out_shape=[
            jax.ShapeDtypeStruct((N, H), jnp.float32),
            jax.ShapeDtypeStruct((2, H), jnp.float32),
        ],
        scratch_shapes=[pltpu.VMEM((2, H), jnp.float32)],
    )(Ah2, Ah2, numden, numden)


def _read_body(hp_ref, st_ref, g_ref, bt_ref, hin_ref, b_ref,
               w1_ref, b1_ref, w2_ref, b2_ref, w3_ref, b3_ref,
               y_ref, hs_ref, cnt_ref):
    i = pl.program_id(0)
    mean = st_ref[0] * (1.0 / N)
    var = st_ref[1] * (1.0 / N) - mean * mean
    rstd = lax.rsqrt(var + 1e-5)
    h = (
        jnp.maximum((hp_ref[...] - mean) * (rstd * g_ref[0]) + bt_ref[0], 0.0)
        + hin_ref[...]
    )
    gid = lax.broadcasted_iota(jnp.int32, (BLK, G), 1)
    oh = (b_ref[...] == gid).astype(jnp.float32)
    dnums = (((0,), (0,)), ((), ()))
    hs = lax.dot_general(oh, h, dnums, preferred_element_type=jnp.float32)
    cn = lax.dot_general(oh, jnp.ones_like(h), dnums,
                         preferred_element_type=jnp.float32)

    @pl.when(i == 0)
    def _():
        hs_ref[...] = hs
        cnt_ref[...] = cn

    @pl.when(i > 0)
    def _():
        hs_ref[...] += hs
        cnt_ref[...] += cn

    @pl.when(i == NB - 1)
    def _():
        hg = hs_ref[...] / jnp.maximum(cnt_ref[...], 1.0)
        y1 = jnp.maximum(
            jnp.dot(hg, w1_ref[...], preferred_element_type=jnp.float32)
            + b1_ref[0], 0.0)
        y2 = jnp.maximum(
            jnp.dot(y1, w2_ref[...], preferred_element_type=jnp.float32)
            + b2_ref[0], 0.0)
        y_ref[...] = (
            jnp.dot(y2, w3_ref[...], preferred_element_type=jnp.float32)
            + b3_ref[0]
        )


def _readout(hpre, stats, gamma, beta, h_in, batch2d, W1, b1, W2, b2, W3p, b3p):
    return pl.pallas_call(
        _read_body,
        grid=(NB,),
        in_specs=[
            pl.BlockSpec((BLK, H), lambda i: (i, 0)),
            pl.BlockSpec((2, H), lambda i: (0, 0)),
            pl.BlockSpec((1, H), lambda i: (0, 0)),
            pl.BlockSpec((1, H), lambda i: (0, 0)),
            pl.BlockSpec((BLK, H), lambda i: (i, 0)),
            pl.BlockSpec((BLK, 1), lambda i: (i, 0)),
            pl.BlockSpec((H, 64), lambda i: (0, 0)),
            pl.BlockSpec((1, 64), lambda i: (0, 0)),
            pl.BlockSpec((64, 32), lambda i: (0, 0)),
            pl.BlockSpec((1, 32), lambda i: (0, 0)),
            pl.BlockSpec((32, H), lambda i: (0, 0)),
            pl.BlockSpec((1, H), lambda i: (0, 0)),
        ],
        out_specs=pl.BlockSpec((G, H), lambda i: (0, 0)),
        out_shape=jax.ShapeDtypeStruct((G, H), jnp.float32),
        scratch_shapes=[
            pltpu.VMEM((G, H), jnp.float32),
            pltpu.VMEM((G, H), jnp.float32),
        ],
    )(hpre, stats, gamma.reshape(1, H), beta.reshape(1, H), h_in, batch2d,
      W1, b1.reshape(1, 64), W2, b2.reshape(1, 32), W3p, b3p.reshape(1, H))


def _edge_body(dh_hbm, eb2_hbm, dsts_hbm, idxd_hbm, idxeb_hbm, out_hbm,
               dst_v0, dst_v1, idd_v0, idd_v1, ideb_v0, ideb_v1,
               dsc_v0, dsc_v1, dr0, dr1, eb0, eb1, ms0, ms1, acc,
               semi0, semi1, semg0, semg1, semh0, semh1, sems0, sems1):
    c = lax.axis_index("c")
    s = lax.axis_index("s")
    dst_v = (dst_v0, dst_v1)
    idd_v = (idd_v0, idd_v1)
    ideb_v = (ideb_v0, ideb_v1)
    dsc_v = (dsc_v0, dsc_v1)
    drows = (dr0, dr1)
    ebrows = (eb0, eb1)
    msgsig = (ms0, ms1)
    semi = (semi0, semi1)
    semg = (semg0, semg1)
    semh = (semh0, semh1)
    sems = (sems0, sems1)

    def _zb16(j, carry):
        r = j // 8
        q = j % 8
        ms0[r, pl.ds(q * 16, 16)] = jnp.zeros((16,), jnp.float32)
        return carry

    lax.fori_loop(0, CH * 8, _zb16, 0)
    row0 = s * RPT
    for t in range(9):
        pltpu.sync_copy(ms0, acc.at[pl.ds(row0 + t * CH, CH)])
    pltpu.sync_copy(ms0.at[pl.ds(0, RPT - 9 * CH)],
                    acc.at[pl.ds(row0 + 9 * CH, RPT - 9 * CH)])

    @pl.when(s == NTEC - 1)
    def _():
        pltpu.sync_copy(ms0.at[pl.ds(0, 24)], acc.at[pl.ds(NTEC * RPT, 24)])

    plsc.subcore_barrier()

    base0 = s * EPT
    coff = c * 64

    def _load_idx_sync(b, base):
        pltpu.sync_copy(dsts_hbm.at[pl.ds(base, CH)], dst_v[b])
        pltpu.sync_copy(idxd_hbm.at[pl.ds(base, CH)], idd_v[b])
        pltpu.sync_copy(idxeb_hbm.at[c, pl.ds(base, CH)], ideb_v[b])

    def _issue_idx(b, base):
        pltpu.async_copy(dsts_hbm.at[pl.ds(base, CH)], dst_v[b], semi[b])
        pltpu.async_copy(idxd_hbm.at[pl.ds(base, CH)], idd_v[b], semi[b])
        pltpu.async_copy(idxeb_hbm.at[c, pl.ds(base, CH)], ideb_v[b], semi[b])

    def _wait_idx(b, base):
        pltpu.make_async_copy(dsts_hbm.at[pl.ds(base, CH)], dst_v[b],
                              semi[b]).wait()
        pltpu.make_async_copy(idxd_hbm.at[pl.ds(base, CH)], idd_v[b],
                              semi[b]).wait()
        pltpu.make_async_copy(idxeb_hbm.at[c, pl.ds(base, CH)], ideb_v[b],
                              semi[b]).wait()

    def _issue_gathers(b):
        pltpu.async_copy(dh_hbm.at[idd_v[b]], drows[b], semg[b])
        pltpu.async_copy(eb2_hbm.at[ideb_v[b]], ebrows[b], semh[b])

    def _wait_gathers(b):
        pltpu.make_async_copy(dh_hbm.at[idd_v[b]], drows[b], semg[b]).wait()
        pltpu.make_async_copy(eb2_hbm.at[ideb_v[b]], ebrows[b],
                              semh[b]).wait()

    def _compute(b):
        dr = drows[b]
        eb = ebrows[b]
        ms = msgsig[b]

        def _row(r, rc):
            for q in range(4):
                d = dr[r, pl.ds(coff + q * 16, 16)]
                e = eb[r, pl.ds(q * 16, 16)]
                sig = 1.0 / (1.0 + jnp.exp(-(d + e)))
                ms[r, pl.ds(q * 16, 16)] = sig
                bb = eb[r, pl.ds(64 + q * 16, 16)]
                ms[r, pl.ds(64 + q * 16, 16)] = sig * bb
            return rc

        lax.fori_loop(0, CH, _row, 0)

    for b in range(2):
        _load_idx_sync(b, base0 + b * CH)
        _issue_gathers(b)

    def _pair(k2, carry):
        for b in range(2):
            k = 2 * k2 + b
            base_next = base0 + (k + 2) * CH
            _wait_gathers(b)

            @pl.when(k >= 2)
            def _():
                pltpu.make_async_copy(msgsig[b], acc.at[dsc_v[b]],
                                      sems[b]).wait()

            for q in range(4):
                dsc_v[b][pl.ds(q * 16, 16)] = dst_v[b][pl.ds(q * 16, 16)]

            @pl.when(k < NCHUNK - 2)
            def _():
                _issue_idx(b, base_next)

            _compute(b)
            pltpu.async_copy(msgsig[b], acc.at[dsc_v[b]], sems[b], add=True)

            @pl.when(k < NCHUNK - 2)
            def _():
                _wait_idx(b, base_next)
                _issue_gathers(b)

        return carry

    lax.fori_loop(0, NPAIR, _pair, 0)

    for b in range(2):
        pltpu.make_async_copy(msgsig[b], acc.at[dsc_v[b]], sems[b]).wait()

    plsc.subcore_barrier()
    pltpu.sync_copy(acc.at[pl.ds(row0, RPT)],
                    out_hbm.at[pl.ds(c * N + row0, RPT)])

    @pl.when(s == NTEC - 1)
    def _():
        pltpu.sync_copy(acc.at[pl.ds(NTEC * RPT, 16)],
                        out_hbm.at[pl.ds(c * N + NTEC * RPT, 16)])


def _edge_pass(Dh, EB2, dst_s, idxD, idxEB):
    mesh = plsc.VectorSubcoreMesh(core_axis_name="c", subcore_axis_name="s",
                                  num_cores=NSC, num_subcores=NTEC)
    f = pl.kernel(
        _edge_body,
        out_type=jax.ShapeDtypeStruct((2 * N, H), jnp.float32),
        mesh=mesh,
        scratch_types=(
            [pltpu.VMEM((CH,), jnp.int32) for _ in range(8)]
            + [pltpu.VMEM((CH, H), jnp.float32) for _ in range(6)]
            + [pltpu.VMEM_SHARED((N + 8, H), jnp.float32)]
            + [pltpu.SemaphoreType.DMA for _ in range(8)]
        ),
    )
    return f(Dh, EB2, dst_s, idxD, idxEB)


def _split_cols(W):
    return jnp.stack([W[:, :, 0:64], W[:, :, 64:128]], axis=1)


def _split_cols_b(b):
    return jnp.stack([b[:, None, 0:64], b[:, None, 64:128]], axis=1)


@jax.jit
def kernel(feature, edge_index, batch, emb_W, emb_b, A_W, A_b, B_W, B_b,
           D_W, D_b, E_W, E_b, bn_gamma, bn_beta, W1, b1, W2, b2, W3, b3):
    src = edge_index[0]
    dst = edge_index[1]
    pad = E2 - E
    zpad = jnp.zeros((pad,), jnp.int32)
    src_g = jnp.concatenate([src, zpad])
    dst_g = jnp.concatenate([dst, zpad])
    dst_s = jnp.concatenate([dst, jnp.full((pad,), N, jnp.int32)])
    idxD = dst_g
    idxEB = jnp.stack([src_g, src_g + N])

    Dfull = jnp.broadcast_to(D_W[:, None], (L, 2, H, H))
    Dfull_b = jnp.broadcast_to(D_b[:, None, None], (L, 2, 1, H))
    Wc = jnp.concatenate(
        [Dfull, _split_cols(A_W), _split_cols(E_W), _split_cols(B_W)],
        axis=-1)
    bcat = jnp.concatenate(
        [Dfull_b, _split_cols_b(A_b), _split_cols_b(E_b), _split_cols_b(B_b)],
        axis=-1)

    W3p = jnp.pad(W3, ((0, 0), (0, H - NCLS)))
    b3p = jnp.pad(b3, (0, H - NCLS))
    batch2d = batch.reshape(N, 1)

    h = _embed(feature, emb_W, emb_b)
    h_in = h
    Ah2, Dh2, EB2 = _mm(h, Wc[0], bcat[0])
    for l in range(L):
        numden = _edge_pass(Dh2, EB2, dst_s, idxD, idxEB)
        hpre, stats = _update(Ah2, numden)
        if l < L - 1:
            h_in, Ah2, Dh2, EB2 = _norm_mm(
                hpre, stats, bn_gamma[l], bn_beta[l], h_in,
                Wc[l + 1], bcat[l + 1])
        else:
            y = _readout(hpre, stats, bn_gamma[l], bn_beta[l], h_in,
                         batch2d, W1, b1, W2, b2, W3p, b3p)
    return y[:, :NCLS]


# speedup vs baseline: 7.1431x; 3.3190x over previous
import functools

import jax
import jax.numpy as jnp
from jax import lax
from jax.experimental import pallas as pl
from jax.experimental.pallas import tpu as pltpu
from jax.experimental.pallas import tpu_sc as plsc

N = 10000
E = 320000
H = 128
L = 4
G = 64
NCLS = 10

NSC = 2
NTEC = 16
CH = 64
NCHUNK = 314
EPT = NCHUNK * CH
E2 = EPT * NTEC
NPAIR = NCHUNK // 2
RPT = 624

BLK = 1000
NB = N // BLK


def _embed_body(x_ref, w_ref, b_ref, o_ref):
    o_ref[...] = (
        jnp.dot(x_ref[...], w_ref[...], preferred_element_type=jnp.float32)
        + b_ref[...]
    )


def _embed(feature, emb_W, emb_b):
    return pl.pallas_call(
        _embed_body,
        grid=(NB,),
        in_specs=[
            pl.BlockSpec((BLK, H), lambda i: (i, 0)),
            pl.BlockSpec((H, H), lambda i: (0, 0)),
            pl.BlockSpec((1, H), lambda i: (0, 0)),
        ],
        out_specs=pl.BlockSpec((BLK, H), lambda i: (i, 0)),
        out_shape=jax.ShapeDtypeStruct((N, H), jnp.float32),
    )(feature, emb_W, emb_b.reshape(1, H))


def _mm_body(h_ref, w_ref, b_ref, a_ref, d_ref, eb_ref):
    out = (
        jnp.dot(h_ref[...], w_ref[0], preferred_element_type=jnp.float32)
        + b_ref[0]
    )
    d_ref[...] = out[:, 0:128]
    a_ref[...] = out[:, 128:192]
    eb_ref[...] = out[:, 192:320]


def _mm(h, Wc, bc):
    return pl.pallas_call(
        _mm_body,
        grid=(NB, 2),
        in_specs=[
            pl.BlockSpec((BLK, H), lambda i, c: (i, 0)),
            pl.BlockSpec((1, H, 5 * 64), lambda i, c: (c, 0, 0)),
            pl.BlockSpec((1, 1, 5 * 64), lambda i, c: (c, 0, 0)),
        ],
        out_specs=[
            pl.BlockSpec((BLK, 64), lambda i, c: (c * NB + i, 0)),
            pl.BlockSpec((BLK, H), lambda i, c: (i, 0)),
            pl.BlockSpec((BLK, H), lambda i, c: (c * NB + i, 0)),
        ],
        out_shape=[
            jax.ShapeDtypeStruct((2 * N, 64), jnp.float32),
            jax.ShapeDtypeStruct((N, H), jnp.float32),
            jax.ShapeDtypeStruct((2 * N, H), jnp.float32),
        ],
    )(h, Wc, bc)


def _norm_mm_body(hp_ref, st_ref, g_ref, bt_ref, hin_ref, w_ref, b_ref,
                  h_ref, a_ref, d_ref, eb_ref):
    mean = st_ref[0] * (1.0 / N)
    var = st_ref[1] * (1.0 / N) - mean * mean
    rstd = lax.rsqrt(var + 1e-5)
    h = (
        jnp.maximum((hp_ref[...] - mean) * (rstd * g_ref[0]) + bt_ref[0], 0.0)
        + hin_ref[...]
    )
    h_ref[...] = h
    out = jnp.dot(h, w_ref[0], preferred_element_type=jnp.float32) + b_ref[0]
    d_ref[...] = out[:, 0:128]
    a_ref[...] = out[:, 128:192]
    eb_ref[...] = out[:, 192:320]


def _norm_mm(hpre, stats, gamma, beta, h_in, Wc, bc):
    return pl.pallas_call(
        _norm_mm_body,
        grid=(NB, 2),
        in_specs=[
            pl.BlockSpec((BLK, H), lambda i, c: (i, 0)),
            pl.BlockSpec((2, H), lambda i, c: (0, 0)),
            pl.BlockSpec((1, H), lambda i, c: (0, 0)),
            pl.BlockSpec((1, H), lambda i, c: (0, 0)),
            pl.BlockSpec((BLK, H), lambda i, c: (i, 0)),
            pl.BlockSpec((1, H, 5 * 64), lambda i, c: (c, 0, 0)),
            pl.BlockSpec((1, 1, 5 * 64), lambda i, c: (c, 0, 0)),
        ],
        out_specs=[
            pl.BlockSpec((BLK, H), lambda i, c: (i, 0)),
            pl.BlockSpec((BLK, 64), lambda i, c: (c * NB + i, 0)),
            pl.BlockSpec((BLK, H), lambda i, c: (i, 0)),
            pl.BlockSpec((BLK, H), lambda i, c: (c * NB + i, 0)),
        ],
        out_shape=[
            jax.ShapeDtypeStruct((N, H), jnp.float32),
            jax.ShapeDtypeStruct((2 * N, 64), jnp.float32),
            jax.ShapeDtypeStruct((N, H), jnp.float32),
            jax.ShapeDtypeStruct((2 * N, H), jnp.float32),
        ],
    )(hpre, stats, gamma.reshape(1, H), beta.reshape(1, H), h_in, Wc, bc)


def _update_body(a0_ref, a1_ref, nd0_ref, nd1_ref, hp_ref, st_ref, acc_ref):
    i = pl.program_id(0)
    nd0 = nd0_ref[...]
    nd1 = nd1_ref[...]
    num = jnp.concatenate([nd0[:, 64:128], nd1[:, 64:128]], axis=1)
    den = jnp.concatenate([nd0[:, 0:64], nd1[:, 0:64]], axis=1)
    ah = jnp.concatenate([a0_ref[...], a1_ref[...]], axis=1)
    hp = ah + num / (den + 1e-6)
    hp_ref[...] = hp
    s1 = jnp.sum(hp, axis=0, keepdims=True)
    s2 = jnp.sum(hp * hp, axis=0, keepdims=True)
    s = jnp.concatenate([s1, s2], axis=0)

    @pl.when(i == 0)
    def _():
        acc_ref[...] = s

    @pl.when(i > 0)
    def _():
        acc_ref[...] += s

    st_ref[...] = acc_ref[...]


def _update(Ah2, numden):
    return pl.pallas_call(
        _update_body,
        grid=(NB,),
        in_specs=[
            pl.BlockSpec((BLK, 64), lambda i: (i, 0)),
            pl.BlockSpec((BLK, 64), lambda i: (NB + i, 0)),
            pl.BlockSpec((BLK, H), lambda i: (i, 0)),
            pl.BlockSpec((BLK, H), lambda i: (NB + i, 0)),
        ],
        out_specs=[
            pl.BlockSpec((BLK, H), lambda i: (i, 0)),
            pl.BlockSpec((2, H), lambda i: (0, 0)),
        ],
        out_shape=[
            jax.ShapeDtypeStruct((N, H), jnp.float32),
            jax.ShapeDtypeStruct((2, H), jnp.float32),
        ],
        scratch_shapes=[pltpu.VMEM((2, H), jnp.float32)],
    )(Ah2, Ah2, numden, numden)


def _read_body(hp_ref, st_ref, g_ref, bt_ref, hin_ref, b_ref,
               w1_ref, b1_ref, w2_ref, b2_ref, w3_ref, b3_ref,
               y_ref, hs_ref, cnt_ref):
    i = pl.program_id(0)
    mean = st_ref[0] * (1.0 / N)
    var = st_ref[1] * (1.0 / N) - mean * mean
    rstd = lax.rsqrt(var + 1e-5)
    h = (
        jnp.maximum((hp_ref[...] - mean) * (rstd * g_ref[0]) + bt_ref[0], 0.0)
        + hin_ref[...]
    )
    gid = lax.broadcasted_iota(jnp.int32, (BLK, G), 1)
    oh = (b_ref[...] == gid).astype(jnp.float32)
    dnums = (((0,), (0,)), ((), ()))
    hs = lax.dot_general(oh, h, dnums, preferred_element_type=jnp.float32)
    cn = lax.dot_general(oh, jnp.ones_like(h), dnums,
                         preferred_element_type=jnp.float32)

    @pl.when(i == 0)
    def _():
        hs_ref[...] = hs
        cnt_ref[...] = cn

    @pl.when(i > 0)
    def _():
        hs_ref[...] += hs
        cnt_ref[...] += cn

    @pl.when(i == NB - 1)
    def _():
        hg = hs_ref[...] / jnp.maximum(cnt_ref[...], 1.0)
        y1 = jnp.maximum(
            jnp.dot(hg, w1_ref[...], preferred_element_type=jnp.float32)
            + b1_ref[0], 0.0)
        y2 = jnp.maximum(
            jnp.dot(y1, w2_ref[...], preferred_element_type=jnp.float32)
            + b2_ref[0], 0.0)
        y_ref[...] = (
            jnp.dot(y2, w3_ref[...], preferred_element_type=jnp.float32)
            + b3_ref[0]
        )


def _readout(hpre, stats, gamma, beta, h_in, batch2d, W1, b1, W2, b2, W3p, b3p):
    return pl.pallas_call(
        _read_body,
        grid=(NB,),
        in_specs=[
            pl.BlockSpec((BLK, H), lambda i: (i, 0)),
            pl.BlockSpec((2, H), lambda i: (0, 0)),
            pl.BlockSpec((1, H), lambda i: (0, 0)),
            pl.BlockSpec((1, H), lambda i: (0, 0)),
            pl.BlockSpec((BLK, H), lambda i: (i, 0)),
            pl.BlockSpec((BLK, 1), lambda i: (i, 0)),
            pl.BlockSpec((H, 64), lambda i: (0, 0)),
            pl.BlockSpec((1, 64), lambda i: (0, 0)),
            pl.BlockSpec((64, 32), lambda i: (0, 0)),
            pl.BlockSpec((1, 32), lambda i: (0, 0)),
            pl.BlockSpec((32, H), lambda i: (0, 0)),
            pl.BlockSpec((1, H), lambda i: (0, 0)),
        ],
        out_specs=pl.BlockSpec((G, H), lambda i: (0, 0)),
        out_shape=jax.ShapeDtypeStruct((G, H), jnp.float32),
        scratch_shapes=[
            pltpu.VMEM((G, H), jnp.float32),
            pltpu.VMEM((G, H), jnp.float32),
        ],
    )(hpre, stats, gamma.reshape(1, H), beta.reshape(1, H), h_in, batch2d,
      W1, b1.reshape(1, 64), W2, b2.reshape(1, 32), W3p, b3p.reshape(1, H))


def _edge_body(dh_hbm, eb2_hbm, dsts_hbm, idxd_hbm, idxeb_hbm, out_hbm,
               dst_v0, dst_v1, idd_v0, idd_v1, ideb_v0, ideb_v1,
               dsc_v0, dsc_v1, dr0, dr1, eb0, eb1, ms0, ms1, acc,
               semi0, semi1, semg0, semg1, semh0, semh1, sems0, sems1):
    c = lax.axis_index("c")
    s = lax.axis_index("s")
    dst_v = (dst_v0, dst_v1)
    idd_v = (idd_v0, idd_v1)
    ideb_v = (ideb_v0, ideb_v1)
    dsc_v = (dsc_v0, dsc_v1)
    drows = (dr0, dr1)
    ebrows = (eb0, eb1)
    msgsig = (ms0, ms1)
    semi = (semi0, semi1)
    semg = (semg0, semg1)
    semh = (semh0, semh1)
    sems = (sems0, sems1)

    def _zb16(j, carry):
        r = j // 8
        q = j % 8
        ms0[r, pl.ds(q * 16, 16)] = jnp.zeros((16,), jnp.float32)
        return carry

    lax.fori_loop(0, CH * 8, _zb16, 0)
    row0 = s * RPT
    for t in range(9):
        pltpu.sync_copy(ms0, acc.at[pl.ds(row0 + t * CH, CH)])
    pltpu.sync_copy(ms0.at[pl.ds(0, RPT - 9 * CH)],
                    acc.at[pl.ds(row0 + 9 * CH, RPT - 9 * CH)])

    @pl.when(s == NTEC - 1)
    def _():
        pltpu.sync_copy(ms0.at[pl.ds(0, 24)], acc.at[pl.ds(NTEC * RPT, 24)])

    plsc.subcore_barrier()

    base0 = s * EPT
    coff = c * 64

    def _load_idx_sync(b, base):
        pltpu.sync_copy(dsts_hbm.at[pl.ds(base, CH)], dst_v[b])
        pltpu.sync_copy(idxd_hbm.at[pl.ds(base, CH)], idd_v[b])
        pltpu.sync_copy(idxeb_hbm.at[c, pl.ds(base, CH)], ideb_v[b])

    def _issue_idx(b, base):
        pltpu.async_copy(dsts_hbm.at[pl.ds(base, CH)], dst_v[b], semi[b])
        pltpu.async_copy(idxd_hbm.at[pl.ds(base, CH)], idd_v[b], semi[b])
        pltpu.async_copy(idxeb_hbm.at[c, pl.ds(base, CH)], ideb_v[b], semi[b])

    def _wait_idx(b, base):
        pltpu.make_async_copy(dsts_hbm.at[pl.ds(base, CH)], dst_v[b],
                              semi[b]).wait()
        pltpu.make_async_copy(idxd_hbm.at[pl.ds(base, CH)], idd_v[b],
                              semi[b]).wait()
        pltpu.make_async_copy(idxeb_hbm.at[c, pl.ds(base, CH)], ideb_v[b],
                              semi[b]).wait()

    def _issue_gathers(b):
        pltpu.async_copy(dh_hbm.at[idd_v[b]], drows[b], semg[b])
        pltpu.async_copy(eb2_hbm.at[ideb_v[b]], ebrows[b], semh[b])

    def _wait_gathers(b):
        pltpu.make_async_copy(dh_hbm.at[idd_v[b]], drows[b], semg[b]).wait()
        pltpu.make_async_copy(eb2_hbm.at[ideb_v[b]], ebrows[b],
                              semh[b]).wait()

    def _compute(b):
        dr = drows[b]
        eb = ebrows[b]
        ms = msgsig[b]

        def _rowpair(rp, rc):
            r0 = 2 * rp
            rows = (r0, r0 + 1)
            xs = [
                dr[r, pl.ds(coff + q * 16, 16)] + eb[r, pl.ds(q * 16, 16)]
                for r in rows for q in range(4)
            ]
            es = [jnp.exp(-x) for x in xs]
            sigs = [1.0 / (1.0 + t) for t in es]
            i = 0
            for r in rows:
                for q in range(4):
                    ms[r, pl.ds(q * 16, 16)] = sigs[i]
                    ms[r, pl.ds(64 + q * 16, 16)] = (
                        sigs[i] * eb[r, pl.ds(64 + q * 16, 16)])
                    i += 1
            return rc

        lax.fori_loop(0, CH // 2, _rowpair, 0)

    for b in range(2):
        _load_idx_sync(b, base0 + b * CH)
        _issue_gathers(b)

    def _pair(k2, carry):
        for b in range(2):
            k = 2 * k2 + b
            base_next = base0 + (k + 2) * CH
            _wait_gathers(b)

            @pl.when(k >= 2)
            def _():
                pltpu.make_async_copy(msgsig[b], acc.at[dsc_v[b]],
                                      sems[b]).wait()

            for q in range(4):
                dsc_v[b][pl.ds(q * 16, 16)] = dst_v[b][pl.ds(q * 16, 16)]

            @pl.when(k < NCHUNK - 2)
            def _():
                _issue_idx(b, base_next)

            _compute(b)
            pltpu.async_copy(msgsig[b], acc.at[dsc_v[b]], sems[b], add=True)

            @pl.when(k < NCHUNK - 2)
            def _():
                _wait_idx(b, base_next)
                _issue_gathers(b)

        return carry

    lax.fori_loop(0, NPAIR, _pair, 0)

    for b in range(2):
        pltpu.make_async_copy(msgsig[b], acc.at[dsc_v[b]], sems[b]).wait()

    plsc.subcore_barrier()
    pltpu.sync_copy(acc.at[pl.ds(row0, RPT)],
                    out_hbm.at[pl.ds(c * N + row0, RPT)])

    @pl.when(s == NTEC - 1)
    def _():
        pltpu.sync_copy(acc.at[pl.ds(NTEC * RPT, 16)],
                        out_hbm.at[pl.ds(c * N + NTEC * RPT, 16)])


def _edge_pass(Dh, EB2, dst_s, idxD, idxEB):
    mesh = plsc.VectorSubcoreMesh(core_axis_name="c", subcore_axis_name="s",
                                  num_cores=NSC, num_subcores=NTEC)
    f = pl.kernel(
        _edge_body,
        out_type=jax.ShapeDtypeStruct((2 * N, H), jnp.float32),
        mesh=mesh,
        scratch_types=(
            [pltpu.VMEM((CH,), jnp.int32) for _ in range(8)]
            + [pltpu.VMEM((CH, H), jnp.float32) for _ in range(6)]
            + [pltpu.VMEM_SHARED((N + 8, H), jnp.float32)]
            + [pltpu.SemaphoreType.DMA for _ in range(8)]
        ),
    )
    return f(Dh, EB2, dst_s, idxD, idxEB)


def _split_cols(W):
    return jnp.stack([W[:, :, 0:64], W[:, :, 64:128]], axis=1)


def _split_cols_b(b):
    return jnp.stack([b[:, None, 0:64], b[:, None, 64:128]], axis=1)


@jax.jit
def kernel(feature, edge_index, batch, emb_W, emb_b, A_W, A_b, B_W, B_b,
           D_W, D_b, E_W, E_b, bn_gamma, bn_beta, W1, b1, W2, b2, W3, b3):
    src = edge_index[0]
    dst = edge_index[1]
    pad = E2 - E
    zpad = jnp.zeros((pad,), jnp.int32)
    src_g = jnp.concatenate([src, zpad])
    dst_g = jnp.concatenate([dst, zpad])
    dst_s = jnp.concatenate([dst, jnp.full((pad,), N, jnp.int32)])
    idxD = dst_g
    idxEB = jnp.stack([src_g, src_g + N])

    Dfull = jnp.broadcast_to(D_W[:, None], (L, 2, H, H))
    Dfull_b = jnp.broadcast_to(D_b[:, None, None], (L, 2, 1, H))
    Wc = jnp.concatenate(
        [Dfull, _split_cols(A_W), _split_cols(E_W), _split_cols(B_W)],
        axis=-1)
    bcat = jnp.concatenate(
        [Dfull_b, _split_cols_b(A_b), _split_cols_b(E_b), _split_cols_b(B_b)],
        axis=-1)

    W3p = jnp.pad(W3, ((0, 0), (0, H - NCLS)))
    b3p = jnp.pad(b3, (0, H - NCLS))
    batch2d = batch.reshape(N, 1)

    h = _embed(feature, emb_W, emb_b)
    h_in = h
    Ah2, Dh2, EB2 = _mm(h, Wc[0], bcat[0])
    for l in range(L):
        numden = _edge_pass(Dh2, EB2, dst_s, idxD, idxEB)
        hpre, stats = _update(Ah2, numden)
        if l < L - 1:
            h_in, Ah2, Dh2, EB2 = _norm_mm(
                hpre, stats, bn_gamma[l], bn_beta[l], h_in,
                Wc[l + 1], bcat[l + 1])
        else:
            y = _readout(hpre, stats, bn_gamma[l], bn_beta[l], h_in,
                         batch2d, W1, b1, W2, b2, W3p, b3p)
    return y[:, :NCLS]


# speedup vs baseline: 7.2155x; 1.0101x over previous
import functools

import jax
import jax.numpy as jnp
from jax import lax
from jax.experimental import pallas as pl
from jax.experimental.pallas import tpu as pltpu
from jax.experimental.pallas import tpu_sc as plsc

N = 10000
E = 320000
H = 128
L = 4
G = 64
NCLS = 10

NSC = 2
NTEC = 16
CH = 64
NCHUNK = 314
EPT = NCHUNK * CH
E2 = EPT * NTEC
NPAIR = NCHUNK // 2
RPT = 624

BLK = 1000
NB = N // BLK


def _embed_body(x_ref, w_ref, b_ref, o_ref):
    o_ref[...] = (
        jnp.dot(x_ref[...], w_ref[...], preferred_element_type=jnp.float32)
        + b_ref[...]
    )


def _embed(feature, emb_W, emb_b):
    return pl.pallas_call(
        _embed_body,
        grid=(NB,),
        in_specs=[
            pl.BlockSpec((BLK, H), lambda i: (i, 0)),
            pl.BlockSpec((H, H), lambda i: (0, 0)),
            pl.BlockSpec((1, H), lambda i: (0, 0)),
        ],
        out_specs=pl.BlockSpec((BLK, H), lambda i: (i, 0)),
        out_shape=jax.ShapeDtypeStruct((N, H), jnp.float32),
    )(feature, emb_W, emb_b.reshape(1, H))


def _mm_body(h_ref, w_ref, b_ref, a_ref, d_ref, eb_ref):
    out = (
        jnp.dot(h_ref[...], w_ref[0], preferred_element_type=jnp.float32)
        + b_ref[0]
    )
    d_ref[...] = out[:, 0:128]
    e16 = lax.bitcast_convert_type(
        out[:, 192:256].astype(jnp.bfloat16), jnp.uint16)
    b16 = lax.bitcast_convert_type(
        out[:, 256:320].astype(jnp.bfloat16), jnp.uint16)
    w = (b16.astype(jnp.uint32) << 16) | e16.astype(jnp.uint32)
    ebp = lax.bitcast_convert_type(w, jnp.float32)
    a_ref[...] = out[:, 128:192]
    eb_ref[...] = jnp.concatenate([ebp, ebp], axis=1)


def _mm(h, Wc, bc):
    return pl.pallas_call(
        _mm_body,
        grid=(NB, 2),
        in_specs=[
            pl.BlockSpec((BLK, H), lambda i, c: (i, 0)),
            pl.BlockSpec((1, H, 5 * 64), lambda i, c: (c, 0, 0)),
            pl.BlockSpec((1, 1, 5 * 64), lambda i, c: (c, 0, 0)),
        ],
        out_specs=[
            pl.BlockSpec((BLK, 64), lambda i, c: (c * NB + i, 0)),
            pl.BlockSpec((BLK, H), lambda i, c: (i, 0)),
            pl.BlockSpec((BLK, H), lambda i, c: (c * NB + i, 0)),
        ],
        out_shape=[
            jax.ShapeDtypeStruct((2 * N, 64), jnp.float32),
            jax.ShapeDtypeStruct((N, H), jnp.float32),
            jax.ShapeDtypeStruct((2 * N, H), jnp.float32),
        ],
    )(h, Wc, bc)


def _norm_mm_body(hp_ref, st_ref, g_ref, bt_ref, hin_ref, w_ref, b_ref,
                  h_ref, a_ref, d_ref, eb_ref):
    mean = st_ref[0] * (1.0 / N)
    var = st_ref[1] * (1.0 / N) - mean * mean
    rstd = lax.rsqrt(var + 1e-5)
    h = (
        jnp.maximum((hp_ref[...] - mean) * (rstd * g_ref[0]) + bt_ref[0], 0.0)
        + hin_ref[...]
    )
    h_ref[...] = h
    out = jnp.dot(h, w_ref[0], preferred_element_type=jnp.float32) + b_ref[0]
    d_ref[...] = out[:, 0:128]
    e16 = lax.bitcast_convert_type(
        out[:, 192:256].astype(jnp.bfloat16), jnp.uint16)
    b16 = lax.bitcast_convert_type(
        out[:, 256:320].astype(jnp.bfloat16), jnp.uint16)
    w = (b16.astype(jnp.uint32) << 16) | e16.astype(jnp.uint32)
    ebp = lax.bitcast_convert_type(w, jnp.float32)
    a_ref[...] = out[:, 128:192]
    eb_ref[...] = jnp.concatenate([ebp, ebp], axis=1)


def _norm_mm(hpre, stats, gamma, beta, h_in, Wc, bc):
    return pl.pallas_call(
        _norm_mm_body,
        grid=(NB, 2),
        in_specs=[
            pl.BlockSpec((BLK, H), lambda i, c: (i, 0)),
            pl.BlockSpec((2, H), lambda i, c: (0, 0)),
            pl.BlockSpec((1, H), lambda i, c: (0, 0)),
            pl.BlockSpec((1, H), lambda i, c: (0, 0)),
            pl.BlockSpec((BLK, H), lambda i, c: (i, 0)),
            pl.BlockSpec((1, H, 5 * 64), lambda i, c: (c, 0, 0)),
            pl.BlockSpec((1, 1, 5 * 64), lambda i, c: (c, 0, 0)),
        ],
        out_specs=[
            pl.BlockSpec((BLK, H), lambda i, c: (i, 0)),
            pl.BlockSpec((BLK, 64), lambda i, c: (c * NB + i, 0)),
            pl.BlockSpec((BLK, H), lambda i, c: (i, 0)),
            pl.BlockSpec((BLK, H), lambda i, c: (c * NB + i, 0)),
        ],
        out_shape=[
            jax.ShapeDtypeStruct((N, H), jnp.float32),
            jax.ShapeDtypeStruct((2 * N, 64), jnp.float32),
            jax.ShapeDtypeStruct((N, H), jnp.float32),
            jax.ShapeDtypeStruct((2 * N, H), jnp.float32),
        ],
    )(hpre, stats, gamma.reshape(1, H), beta.reshape(1, H), h_in, Wc, bc)


def _update_body(a0_ref, a1_ref, nd0_ref, nd1_ref, hp_ref, st_ref, acc_ref):
    i = pl.program_id(0)
    nd0 = nd0_ref[...]
    nd1 = nd1_ref[...]
    num = jnp.concatenate([nd0[:, 64:128], nd1[:, 64:128]], axis=1)
    den = jnp.concatenate([nd0[:, 0:64], nd1[:, 0:64]], axis=1)
    ah = jnp.concatenate([a0_ref[...], a1_ref[...]], axis=1)
    hp = ah + num / (den + 1e-6)
    hp_ref[...] = hp
    s1 = jnp.sum(hp, axis=0, keepdims=True)
    s2 = jnp.sum(hp * hp, axis=0, keepdims=True)
    s = jnp.concatenate([s1, s2], axis=0)

    @pl.when(i == 0)
    def _():
        acc_ref[...] = s

    @pl.when(i > 0)
    def _():
        acc_ref[...] += s

    st_ref[...] = acc_ref[...]


def _update(Ah2, numden):
    return pl.pallas_call(
        _update_body,
        grid=(NB,),
        in_specs=[
            pl.BlockSpec((BLK, 64), lambda i: (i, 0)),
            pl.BlockSpec((BLK, 64), lambda i: (NB + i, 0)),
            pl.BlockSpec((BLK, H), lambda i: (i, 0)),
            pl.BlockSpec((BLK, H), lambda i: (NB + i, 0)),
        ],
        out_specs=[
            pl.BlockSpec((BLK, H), lambda i: (i, 0)),
            pl.BlockSpec((2, H), lambda i: (0, 0)),
        ],
        out_shape=[
            jax.ShapeDtypeStruct((N, H), jnp.float32),
            jax.ShapeDtypeStruct((2, H), jnp.float32),
        ],
        scratch_shapes=[pltpu.VMEM((2, H), jnp.float32)],
    )(Ah2, Ah2, numden, numden)


def _read_body(hp_ref, st_ref, g_ref, bt_ref, hin_ref, b_ref,
               w1_ref, b1_ref, w2_ref, b2_ref, w3_ref, b3_ref,
               y_ref, hs_ref, cnt_ref):
    i = pl.program_id(0)
    mean = st_ref[0] * (1.0 / N)
    var = st_ref[1] * (1.0 / N) - mean * mean
    rstd = lax.rsqrt(var + 1e-5)
    h = (
        jnp.maximum((hp_ref[...] - mean) * (rstd * g_ref[0]) + bt_ref[0], 0.0)
        + hin_ref[...]
    )
    gid = lax.broadcasted_iota(jnp.int32, (BLK, G), 1)
    oh = (b_ref[...] == gid).astype(jnp.float32)
    dnums = (((0,), (0,)), ((), ()))
    hs = lax.dot_general(oh, h, dnums, preferred_element_type=jnp.float32)
    cn = lax.dot_general(oh, jnp.ones_like(h), dnums,
                         preferred_element_type=jnp.float32)

    @pl.when(i == 0)
    def _():
        hs_ref[...] = hs
        cnt_ref[...] = cn

    @pl.when(i > 0)
    def _():
        hs_ref[...] += hs
        cnt_ref[...] += cn

    @pl.when(i == NB - 1)
    def _():
        hg = hs_ref[...] / jnp.maximum(cnt_ref[...], 1.0)
        y1 = jnp.maximum(
            jnp.dot(hg, w1_ref[...], preferred_element_type=jnp.float32)
            + b1_ref[0], 0.0)
        y2 = jnp.maximum(
            jnp.dot(y1, w2_ref[...], preferred_element_type=jnp.float32)
            + b2_ref[0], 0.0)
        y_ref[...] = (
            jnp.dot(y2, w3_ref[...], preferred_element_type=jnp.float32)
            + b3_ref[0]
        )


def _readout(hpre, stats, gamma, beta, h_in, batch2d, W1, b1, W2, b2, W3p, b3p):
    return pl.pallas_call(
        _read_body,
        grid=(NB,),
        in_specs=[
            pl.BlockSpec((BLK, H), lambda i: (i, 0)),
            pl.BlockSpec((2, H), lambda i: (0, 0)),
            pl.BlockSpec((1, H), lambda i: (0, 0)),
            pl.BlockSpec((1, H), lambda i: (0, 0)),
            pl.BlockSpec((BLK, H), lambda i: (i, 0)),
            pl.BlockSpec((BLK, 1), lambda i: (i, 0)),
            pl.BlockSpec((H, 64), lambda i: (0, 0)),
            pl.BlockSpec((1, 64), lambda i: (0, 0)),
            pl.BlockSpec((64, 32), lambda i: (0, 0)),
            pl.BlockSpec((1, 32), lambda i: (0, 0)),
            pl.BlockSpec((32, H), lambda i: (0, 0)),
            pl.BlockSpec((1, H), lambda i: (0, 0)),
        ],
        out_specs=pl.BlockSpec((G, H), lambda i: (0, 0)),
        out_shape=jax.ShapeDtypeStruct((G, H), jnp.float32),
        scratch_shapes=[
            pltpu.VMEM((G, H), jnp.float32),
            pltpu.VMEM((G, H), jnp.float32),
        ],
    )(hpre, stats, gamma.reshape(1, H), beta.reshape(1, H), h_in, batch2d,
      W1, b1.reshape(1, 64), W2, b2.reshape(1, 32), W3p, b3p.reshape(1, H))


def _edge_body(dh_hbm, eb2_hbm, dsts_hbm, idxd_hbm, idxeb_hbm, out_hbm,
               dst_v0, dst_v1, idd_v0, idd_v1, ideb_v0, ideb_v1,
               dsc_v0, dsc_v1, dr0, dr1, eb0, eb1, ms0, ms1, acc,
               semi0, semi1, semg0, semg1, semh0, semh1, sems0, sems1):
    c = lax.axis_index("c")
    s = lax.axis_index("s")
    dst_v = (dst_v0, dst_v1)
    idd_v = (idd_v0, idd_v1)
    ideb_v = (ideb_v0, ideb_v1)
    dsc_v = (dsc_v0, dsc_v1)
    drows = (dr0, dr1)
    ebrows = (eb0, eb1)
    msgsig = (ms0, ms1)
    semi = (semi0, semi1)
    semg = (semg0, semg1)
    semh = (semh0, semh1)
    sems = (sems0, sems1)

    def _zb16(j, carry):
        r = j // 8
        q = j % 8
        ms0[r, pl.ds(q * 16, 16)] = jnp.zeros((16,), jnp.float32)
        return carry

    lax.fori_loop(0, CH * 8, _zb16, 0)
    row0 = s * RPT
    for t in range(9):
        pltpu.sync_copy(ms0, acc.at[pl.ds(row0 + t * CH, CH)])
    pltpu.sync_copy(ms0.at[pl.ds(0, RPT - 9 * CH)],
                    acc.at[pl.ds(row0 + 9 * CH, RPT - 9 * CH)])

    @pl.when(s == NTEC - 1)
    def _():
        pltpu.sync_copy(ms0.at[pl.ds(0, 24)], acc.at[pl.ds(NTEC * RPT, 24)])

    plsc.subcore_barrier()

    base0 = s * EPT
    coff = c * 64

    def _load_idx_sync(b, base):
        pltpu.sync_copy(dsts_hbm.at[pl.ds(base, CH)], dst_v[b])
        pltpu.sync_copy(idxd_hbm.at[pl.ds(base, CH)], idd_v[b])
        pltpu.sync_copy(idxeb_hbm.at[c, pl.ds(base, CH)], ideb_v[b])

    def _issue_idx(b, base):
        pltpu.async_copy(dsts_hbm.at[pl.ds(base, CH)], dst_v[b], semi[b])
        pltpu.async_copy(idxd_hbm.at[pl.ds(base, CH)], idd_v[b], semi[b])
        pltpu.async_copy(idxeb_hbm.at[c, pl.ds(base, CH)], ideb_v[b], semi[b])

    def _wait_idx(b, base):
        pltpu.make_async_copy(dsts_hbm.at[pl.ds(base, CH)], dst_v[b],
                              semi[b]).wait()
        pltpu.make_async_copy(idxd_hbm.at[pl.ds(base, CH)], idd_v[b],
                              semi[b]).wait()
        pltpu.make_async_copy(idxeb_hbm.at[c, pl.ds(base, CH)], ideb_v[b],
                              semi[b]).wait()

    def _issue_gathers(b):
        pltpu.async_copy(dh_hbm.at[idd_v[b]], drows[b], semg[b])
        pltpu.async_copy(eb2_hbm.at[ideb_v[b]], ebrows[b], semh[b])

    def _wait_gathers(b):
        pltpu.make_async_copy(dh_hbm.at[idd_v[b]], drows[b], semg[b]).wait()
        pltpu.make_async_copy(eb2_hbm.at[ideb_v[b]], ebrows[b],
                              semh[b]).wait()

    def _compute(b):
        dr = drows[b]
        eb = ebrows[b]
        ms = msgsig[b]

        def _quad(rq, rc):
            r0 = 4 * rq
            vals = []
            for r_ in range(4):
                r = r0 + r_
                for q in range(4):
                    d = dr[r, pl.ds(coff + q * 16, 16)]
                    w = plsc.bitcast(eb[r, pl.ds(q * 16, 16)], jnp.int32)
                    e = plsc.bitcast(w << 16, jnp.float32)
                    bb = plsc.bitcast(w & jnp.int32(-65536), jnp.float32)
                    vals.append((d + e, bb))
            es = [jnp.exp(-x) for x, _ in vals]
            sigs = [1.0 / (1.0 + t_) for t_ in es]
            k = 0
            for r_ in range(4):
                r = r0 + r_
                for q in range(4):
                    ms[r, pl.ds(q * 16, 16)] = sigs[k]
                    ms[r, pl.ds(64 + q * 16, 16)] = sigs[k] * vals[k][1]
                    k += 1
            return rc

        lax.fori_loop(0, CH // 4, _quad, 0)

    for b in range(2):
        _load_idx_sync(b, base0 + b * CH)
        _issue_gathers(b)

    def _pair(k2, carry):
        for b in range(2):
            k = 2 * k2 + b
            base_next = base0 + (k + 2) * CH
            _wait_gathers(b)

            @pl.when(k >= 2)
            def _():
                pltpu.make_async_copy(msgsig[b], acc.at[dsc_v[b]],
                                      sems[b]).wait()

            for q in range(4):
                dsc_v[b][pl.ds(q * 16, 16)] = dst_v[b][pl.ds(q * 16, 16)]

            @pl.when(k < NCHUNK - 2)
            def _():
                _issue_idx(b, base_next)

            _compute(b)
            pltpu.async_copy(msgsig[b], acc.at[dsc_v[b]], sems[b], add=True)

            @pl.when(k < NCHUNK - 2)
            def _():
                _wait_idx(b, base_next)
                _issue_gathers(b)

        return carry

    lax.fori_loop(0, NPAIR, _pair, 0)

    for b in range(2):
        pltpu.make_async_copy(msgsig[b], acc.at[dsc_v[b]], sems[b]).wait()

    plsc.subcore_barrier()
    pltpu.sync_copy(acc.at[pl.ds(row0, RPT)],
                    out_hbm.at[pl.ds(c * N + row0, RPT)])

    @pl.when(s == NTEC - 1)
    def _():
        pltpu.sync_copy(acc.at[pl.ds(NTEC * RPT, 16)],
                        out_hbm.at[pl.ds(c * N + NTEC * RPT, 16)])


def _edge_pass(Dh, EB2, dst_s, idxD, idxEB):
    mesh = plsc.VectorSubcoreMesh(core_axis_name="c", subcore_axis_name="s",
                                  num_cores=NSC, num_subcores=NTEC)
    f = pl.kernel(
        _edge_body,
        out_type=jax.ShapeDtypeStruct((2 * N, H), jnp.float32),
        mesh=mesh,
        compiler_params=pltpu.CompilerParams(needs_layout_passes=False),
        scratch_types=(
            [pltpu.VMEM((CH,), jnp.int32) for _ in range(8)]
            + [pltpu.VMEM((CH, H), jnp.float32) for _ in range(6)]
            + [pltpu.VMEM_SHARED((N + 8, H), jnp.float32)]
            + [pltpu.SemaphoreType.DMA for _ in range(8)]
        ),
    )
    return f(Dh, EB2, dst_s, idxD, idxEB)


_PERM128 = tuple(32 * (t // 32) + (t % 32) // 2 + 16 * (t % 2)
                 for t in range(128))
_PERM64 = _PERM128[:64]


def _split_cols(W, perm=None):
    h0, h1 = W[:, :, 0:64], W[:, :, 64:128]
    if perm is not None:
        p = jnp.asarray(perm)
        h0, h1 = h0[:, :, p], h1[:, :, p]
    return jnp.stack([h0, h1], axis=1)


def _split_cols_b(b, perm=None):
    h0, h1 = b[:, None, 0:64], b[:, None, 64:128]
    if perm is not None:
        p = jnp.asarray(perm)
        h0, h1 = h0[:, :, p], h1[:, :, p]
    return jnp.stack([h0, h1], axis=1)


@jax.jit
def kernel(feature, edge_index, batch, emb_W, emb_b, A_W, A_b, B_W, B_b,
           D_W, D_b, E_W, E_b, bn_gamma, bn_beta, W1, b1, W2, b2, W3, b3):
    src = edge_index[0]
    dst = edge_index[1]
    pad = E2 - E
    zpad = jnp.zeros((pad,), jnp.int32)
    src_g = jnp.concatenate([src, zpad])
    dst_g = jnp.concatenate([dst, zpad])
    dst_s = jnp.concatenate([dst, jnp.full((pad,), N, jnp.int32)])
    idxD = dst_g
    idxEB = jnp.stack([src_g, src_g + N])

    Dfull = jnp.broadcast_to(D_W[:, None], (L, 2, H, H))
    Dfull_b = jnp.broadcast_to(D_b[:, None, None], (L, 2, 1, H))
    Wc = jnp.concatenate(
        [Dfull, _split_cols(A_W), _split_cols(E_W), _split_cols(B_W)],
        axis=-1)
    bcat = jnp.concatenate(
        [Dfull_b, _split_cols_b(A_b), _split_cols_b(E_b), _split_cols_b(B_b)],
        axis=-1)

    W3p = jnp.pad(W3, ((0, 0), (0, H - NCLS)))
    b3p = jnp.pad(b3, (0, H - NCLS))
    batch2d = batch.reshape(N, 1)

    h = _embed(feature, emb_W, emb_b)
    h_in = h
    Ah2, Dh2, EB2 = _mm(h, Wc[0], bcat[0])
    for l in range(L):
        numden = _edge_pass(Dh2, EB2, dst_s, idxD, idxEB)
        hpre, stats = _update(Ah2, numden)
        if l < L - 1:
            h_in, Ah2, Dh2, EB2 = _norm_mm(
                hpre, stats, bn_gamma[l], bn_beta[l], h_in,
                Wc[l + 1], bcat[l + 1])
        else:
            y = _readout(hpre, stats, bn_gamma[l], bn_beta[l], h_in,
                         batch2d, W1, b1, W2, b2, W3p, b3p)
    return y[:, :NCLS]


# speedup vs baseline: 7.2609x; 1.0063x over previous
import functools

import jax
import jax.numpy as jnp
from jax import lax
from jax.experimental import pallas as pl
from jax.experimental.pallas import tpu as pltpu
from jax.experimental.pallas import tpu_sc as plsc

N = 10000
E = 320000
H = 128
L = 4
G = 64
NCLS = 10

NSC = 2
NTEC = 16
CH = 64
NCHUNK = 314
EPT = NCHUNK * CH
E2 = EPT * NTEC
NPAIR = NCHUNK // 2
RPT = 624

BLK = 1000
NB = N // BLK


def _embed_body(x_ref, w_ref, b_ref, o_ref):
    o_ref[...] = (
        jnp.dot(x_ref[...], w_ref[...], preferred_element_type=jnp.float32)
        + b_ref[...]
    )


def _embed(feature, emb_W, emb_b):
    return pl.pallas_call(
        _embed_body,
        grid=(NB,),
        in_specs=[
            pl.BlockSpec((BLK, H), lambda i: (i, 0)),
            pl.BlockSpec((H, H), lambda i: (0, 0)),
            pl.BlockSpec((1, H), lambda i: (0, 0)),
        ],
        out_specs=pl.BlockSpec((BLK, H), lambda i: (i, 0)),
        out_shape=jax.ShapeDtypeStruct((N, H), jnp.float32),
    )(feature, emb_W, emb_b.reshape(1, H))


def _mm_body(h_ref, w_ref, b_ref, a_ref, d_ref, eb_ref):
    out = (
        jnp.dot(h_ref[...], w_ref[0], preferred_element_type=jnp.float32)
        + b_ref[0]
    )
    d_ref[...] = out[:, 0:128]
    e16 = lax.bitcast_convert_type(
        out[:, 192:256].astype(jnp.bfloat16), jnp.uint16)
    b16 = lax.bitcast_convert_type(
        out[:, 256:320].astype(jnp.bfloat16), jnp.uint16)
    w = (b16.astype(jnp.uint32) << 16) | e16.astype(jnp.uint32)
    ebp = lax.bitcast_convert_type(w, jnp.float32)
    a_ref[...] = out[:, 128:192]
    eb_ref[...] = jnp.concatenate([ebp, ebp], axis=1)


def _mm(h, Wc, bc):
    return pl.pallas_call(
        _mm_body,
        grid=(NB, 2),
        in_specs=[
            pl.BlockSpec((BLK, H), lambda i, c: (i, 0)),
            pl.BlockSpec((1, H, 5 * 64), lambda i, c: (c, 0, 0)),
            pl.BlockSpec((1, 1, 5 * 64), lambda i, c: (c, 0, 0)),
        ],
        out_specs=[
            pl.BlockSpec((BLK, 64), lambda i, c: (c * NB + i, 0)),
            pl.BlockSpec((BLK, H), lambda i, c: (i, 0)),
            pl.BlockSpec((BLK, H), lambda i, c: (c * NB + i, 0)),
        ],
        out_shape=[
            jax.ShapeDtypeStruct((2 * N, 64), jnp.float32),
            jax.ShapeDtypeStruct((N, H), jnp.float32),
            jax.ShapeDtypeStruct((2 * N, H), jnp.float32),
        ],
    )(h, Wc, bc)


def _norm_mm_body(hp_ref, st_ref, g_ref, bt_ref, hin_ref, w_ref, b_ref,
                  h_ref, a_ref, d_ref, eb_ref):
    mean = st_ref[0] * (1.0 / N)
    var = st_ref[1] * (1.0 / N) - mean * mean
    rstd = lax.rsqrt(var + 1e-5)
    h = (
        jnp.maximum((hp_ref[...] - mean) * (rstd * g_ref[0]) + bt_ref[0], 0.0)
        + hin_ref[...]
    )
    h_ref[...] = h
    out = jnp.dot(h, w_ref[0], preferred_element_type=jnp.float32) + b_ref[0]
    d_ref[...] = out[:, 0:128]
    e16 = lax.bitcast_convert_type(
        out[:, 192:256].astype(jnp.bfloat16), jnp.uint16)
    b16 = lax.bitcast_convert_type(
        out[:, 256:320].astype(jnp.bfloat16), jnp.uint16)
    w = (b16.astype(jnp.uint32) << 16) | e16.astype(jnp.uint32)
    ebp = lax.bitcast_convert_type(w, jnp.float32)
    a_ref[...] = out[:, 128:192]
    eb_ref[...] = jnp.concatenate([ebp, ebp], axis=1)


def _norm_mm(hpre, stats, gamma, beta, h_in, Wc, bc):
    return pl.pallas_call(
        _norm_mm_body,
        grid=(NB, 2),
        in_specs=[
            pl.BlockSpec((BLK, H), lambda i, c: (i, 0)),
            pl.BlockSpec((2, H), lambda i, c: (0, 0)),
            pl.BlockSpec((1, H), lambda i, c: (0, 0)),
            pl.BlockSpec((1, H), lambda i, c: (0, 0)),
            pl.BlockSpec((BLK, H), lambda i, c: (i, 0)),
            pl.BlockSpec((1, H, 5 * 64), lambda i, c: (c, 0, 0)),
            pl.BlockSpec((1, 1, 5 * 64), lambda i, c: (c, 0, 0)),
        ],
        out_specs=[
            pl.BlockSpec((BLK, H), lambda i, c: (i, 0)),
            pl.BlockSpec((BLK, 64), lambda i, c: (c * NB + i, 0)),
            pl.BlockSpec((BLK, H), lambda i, c: (i, 0)),
            pl.BlockSpec((BLK, H), lambda i, c: (c * NB + i, 0)),
        ],
        out_shape=[
            jax.ShapeDtypeStruct((N, H), jnp.float32),
            jax.ShapeDtypeStruct((2 * N, 64), jnp.float32),
            jax.ShapeDtypeStruct((N, H), jnp.float32),
            jax.ShapeDtypeStruct((2 * N, H), jnp.float32),
        ],
    )(hpre, stats, gamma.reshape(1, H), beta.reshape(1, H), h_in, Wc, bc)


def _update_body(a0_ref, a1_ref, nd0_ref, nd1_ref, hp_ref, st_ref, acc_ref):
    i = pl.program_id(0)
    nd0 = nd0_ref[...]
    nd1 = nd1_ref[...]
    num = jnp.concatenate([nd0[:, 64:128], nd1[:, 64:128]], axis=1)
    den = jnp.concatenate([nd0[:, 0:64], nd1[:, 0:64]], axis=1)
    ah = jnp.concatenate([a0_ref[...], a1_ref[...]], axis=1)
    hp = ah + num / (den + 1e-6)
    hp_ref[...] = hp
    s1 = jnp.sum(hp, axis=0, keepdims=True)
    s2 = jnp.sum(hp * hp, axis=0, keepdims=True)
    s = jnp.concatenate([s1, s2], axis=0)

    @pl.when(i == 0)
    def _():
        acc_ref[...] = s

    @pl.when(i > 0)
    def _():
        acc_ref[...] += s

    st_ref[...] = acc_ref[...]


def _update(Ah2, numden):
    return pl.pallas_call(
        _update_body,
        grid=(NB,),
        in_specs=[
            pl.BlockSpec((BLK, 64), lambda i: (i, 0)),
            pl.BlockSpec((BLK, 64), lambda i: (NB + i, 0)),
            pl.BlockSpec((BLK, H), lambda i: (i, 0)),
            pl.BlockSpec((BLK, H), lambda i: (NB + i, 0)),
        ],
        out_specs=[
            pl.BlockSpec((BLK, H), lambda i: (i, 0)),
            pl.BlockSpec((2, H), lambda i: (0, 0)),
        ],
        out_shape=[
            jax.ShapeDtypeStruct((N, H), jnp.float32),
            jax.ShapeDtypeStruct((2, H), jnp.float32),
        ],
        scratch_shapes=[pltpu.VMEM((2, H), jnp.float32)],
    )(Ah2, Ah2, numden, numden)


def _read_body(hp_ref, st_ref, g_ref, bt_ref, hin_ref, b_ref,
               w1_ref, b1_ref, w2_ref, b2_ref, w3_ref, b3_ref,
               y_ref, hs_ref, cnt_ref):
    i = pl.program_id(0)
    mean = st_ref[0] * (1.0 / N)
    var = st_ref[1] * (1.0 / N) - mean * mean
    rstd = lax.rsqrt(var + 1e-5)
    h = (
        jnp.maximum((hp_ref[...] - mean) * (rstd * g_ref[0]) + bt_ref[0], 0.0)
        + hin_ref[...]
    )
    gid = lax.broadcasted_iota(jnp.int32, (BLK, G), 1)
    oh = (b_ref[...] == gid).astype(jnp.float32)
    dnums = (((0,), (0,)), ((), ()))
    hs = lax.dot_general(oh, h, dnums, preferred_element_type=jnp.float32)
    cn = lax.dot_general(oh, jnp.ones_like(h), dnums,
                         preferred_element_type=jnp.float32)

    @pl.when(i == 0)
    def _():
        hs_ref[...] = hs
        cnt_ref[...] = cn

    @pl.when(i > 0)
    def _():
        hs_ref[...] += hs
        cnt_ref[...] += cn

    @pl.when(i == NB - 1)
    def _():
        hg = hs_ref[...] / jnp.maximum(cnt_ref[...], 1.0)
        y1 = jnp.maximum(
            jnp.dot(hg, w1_ref[...], preferred_element_type=jnp.float32)
            + b1_ref[0], 0.0)
        y2 = jnp.maximum(
            jnp.dot(y1, w2_ref[...], preferred_element_type=jnp.float32)
            + b2_ref[0], 0.0)
        y_ref[...] = (
            jnp.dot(y2, w3_ref[...], preferred_element_type=jnp.float32)
            + b3_ref[0]
        )


def _readout(hpre, stats, gamma, beta, h_in, batch2d, W1, b1, W2, b2, W3p, b3p):
    return pl.pallas_call(
        _read_body,
        grid=(NB,),
        in_specs=[
            pl.BlockSpec((BLK, H), lambda i: (i, 0)),
            pl.BlockSpec((2, H), lambda i: (0, 0)),
            pl.BlockSpec((1, H), lambda i: (0, 0)),
            pl.BlockSpec((1, H), lambda i: (0, 0)),
            pl.BlockSpec((BLK, H), lambda i: (i, 0)),
            pl.BlockSpec((BLK, 1), lambda i: (i, 0)),
            pl.BlockSpec((H, 64), lambda i: (0, 0)),
            pl.BlockSpec((1, 64), lambda i: (0, 0)),
            pl.BlockSpec((64, 32), lambda i: (0, 0)),
            pl.BlockSpec((1, 32), lambda i: (0, 0)),
            pl.BlockSpec((32, H), lambda i: (0, 0)),
            pl.BlockSpec((1, H), lambda i: (0, 0)),
        ],
        out_specs=pl.BlockSpec((G, H), lambda i: (0, 0)),
        out_shape=jax.ShapeDtypeStruct((G, H), jnp.float32),
        scratch_shapes=[
            pltpu.VMEM((G, H), jnp.float32),
            pltpu.VMEM((G, H), jnp.float32),
        ],
    )(hpre, stats, gamma.reshape(1, H), beta.reshape(1, H), h_in, batch2d,
      W1, b1.reshape(1, 64), W2, b2.reshape(1, 32), W3p, b3p.reshape(1, H))


def _edge_body(dh_hbm, eb2_hbm, dsts_hbm, idxd_hbm, idxeb_hbm, out_hbm,
               dst_v0, dst_v1, idd_v0, idd_v1, ideb_v0, ideb_v1,
               dsc_v0, dsc_v1, dr0, dr1, eb0, eb1, ms0, ms1, acc,
               semi0, semi1, semg0, semg1, semh0, semh1, sems0, sems1):
    c = lax.axis_index("c")
    s = lax.axis_index("s")
    dst_v = (dst_v0, dst_v1)
    idd_v = (idd_v0, idd_v1)
    ideb_v = (ideb_v0, ideb_v1)
    dsc_v = (dsc_v0, dsc_v1)
    drows = (dr0, dr1)
    ebrows = (eb0, eb1)
    msgsig = (ms0, ms1)
    semi = (semi0, semi1)
    semg = (semg0, semg1)
    semh = (semh0, semh1)
    sems = (sems0, sems1)

    def _zb16(j, carry):
        r = j // 8
        q = j % 8
        ms0[r, pl.ds(q * 16, 16)] = jnp.zeros((16,), jnp.float32)
        return carry

    lax.fori_loop(0, CH * 8, _zb16, 0)
    row0 = s * RPT
    for t in range(9):
        pltpu.sync_copy(ms0, acc.at[pl.ds(row0 + t * CH, CH)])
    pltpu.sync_copy(ms0.at[pl.ds(0, RPT - 9 * CH)],
                    acc.at[pl.ds(row0 + 9 * CH, RPT - 9 * CH)])

    @pl.when(s == NTEC - 1)
    def _():
        pltpu.sync_copy(ms0.at[pl.ds(0, 24)], acc.at[pl.ds(NTEC * RPT, 24)])

    plsc.subcore_barrier()

    base0 = s * EPT
    coff = c * 64

    def _load_idx_sync(b, base):
        pltpu.sync_copy(dsts_hbm.at[pl.ds(base, CH)], dst_v[b])
        pltpu.sync_copy(idxd_hbm.at[pl.ds(base, CH)], idd_v[b])
        pltpu.sync_copy(idxeb_hbm.at[c, pl.ds(base, CH)], ideb_v[b])

    def _issue_idx(b, base):
        pltpu.async_copy(dsts_hbm.at[pl.ds(base, CH)], dst_v[b], semi[b])
        pltpu.async_copy(idxd_hbm.at[pl.ds(base, CH)], idd_v[b], semi[b])
        pltpu.async_copy(idxeb_hbm.at[c, pl.ds(base, CH)], ideb_v[b], semi[b])

    def _wait_idx(b, base):
        pltpu.make_async_copy(dsts_hbm.at[pl.ds(base, CH)], dst_v[b],
                              semi[b]).wait()
        pltpu.make_async_copy(idxd_hbm.at[pl.ds(base, CH)], idd_v[b],
                              semi[b]).wait()
        pltpu.make_async_copy(idxeb_hbm.at[c, pl.ds(base, CH)], ideb_v[b],
                              semi[b]).wait()

    def _issue_gathers(b):
        pltpu.async_copy(dh_hbm.at[idd_v[b]], drows[b], semg[b])
        pltpu.async_copy(eb2_hbm.at[ideb_v[b]], ebrows[b], semh[b])

    def _wait_gathers(b):
        pltpu.make_async_copy(dh_hbm.at[idd_v[b]], drows[b], semg[b]).wait()
        pltpu.make_async_copy(eb2_hbm.at[ideb_v[b]], ebrows[b],
                              semh[b]).wait()

    def _compute(b):
        dr = drows[b]
        eb = ebrows[b]
        ms = msgsig[b]

        def _quad(rq, rc):
            r0 = 4 * rq
            vals = []
            for r_ in range(4):
                r = r0 + r_
                for q in range(4):
                    d = dr[r, pl.ds(coff + q * 16, 16)]
                    w = plsc.bitcast(eb[r, pl.ds(q * 16, 16)], jnp.int32)
                    e = plsc.bitcast(w << 16, jnp.float32)
                    bb = plsc.bitcast(w & jnp.int32(-65536), jnp.float32)
                    vals.append((d + e, bb))
            es = [jnp.exp(x) for x, _ in vals]
            sigs = [1.0 / (1.0 + t_) for t_ in es]
            k = 0
            for r_ in range(4):
                r = r0 + r_
                for q in range(4):
                    ms[r, pl.ds(q * 16, 16)] = sigs[k]
                    ms[r, pl.ds(64 + q * 16, 16)] = sigs[k] * vals[k][1]
                    k += 1
            return rc

        lax.fori_loop(0, CH // 4, _quad, 0)

    for b in range(2):
        _load_idx_sync(b, base0 + b * CH)
        _issue_gathers(b)

    def _pair(k2, carry):
        for b in range(2):
            k = 2 * k2 + b
            base_next = base0 + (k + 2) * CH
            _wait_gathers(b)

            @pl.when(k >= 2)
            def _():
                pltpu.make_async_copy(msgsig[b], acc.at[dsc_v[b]],
                                      sems[b]).wait()

            for q in range(4):
                dsc_v[b][pl.ds(q * 16, 16)] = dst_v[b][pl.ds(q * 16, 16)]

            @pl.when(k < NCHUNK - 2)
            def _():
                _issue_idx(b, base_next)

            _compute(b)
            pltpu.async_copy(msgsig[b], acc.at[dsc_v[b]], sems[b], add=True)

            @pl.when(k < NCHUNK - 2)
            def _():
                _wait_idx(b, base_next)
                _issue_gathers(b)

        return carry

    lax.fori_loop(0, NPAIR, _pair, 0)

    for b in range(2):
        pltpu.make_async_copy(msgsig[b], acc.at[dsc_v[b]], sems[b]).wait()

    plsc.subcore_barrier()
    pltpu.sync_copy(acc.at[pl.ds(row0, RPT)],
                    out_hbm.at[pl.ds(c * N + row0, RPT)])

    @pl.when(s == NTEC - 1)
    def _():
        pltpu.sync_copy(acc.at[pl.ds(NTEC * RPT, 16)],
                        out_hbm.at[pl.ds(c * N + NTEC * RPT, 16)])


def _edge_pass(Dh, EB2, dst_s, idxD, idxEB):
    mesh = plsc.VectorSubcoreMesh(core_axis_name="c", subcore_axis_name="s",
                                  num_cores=NSC, num_subcores=NTEC)
    f = pl.kernel(
        _edge_body,
        out_type=jax.ShapeDtypeStruct((2 * N, H), jnp.float32),
        mesh=mesh,
        compiler_params=pltpu.CompilerParams(needs_layout_passes=False),
        scratch_types=(
            [pltpu.VMEM((CH,), jnp.int32) for _ in range(8)]
            + [pltpu.VMEM((CH, H), jnp.float32) for _ in range(6)]
            + [pltpu.VMEM_SHARED((N + 8, H), jnp.float32)]
            + [pltpu.SemaphoreType.DMA for _ in range(8)]
        ),
    )
    return f(Dh, EB2, dst_s, idxD, idxEB)


_PERM128 = tuple(32 * (t // 32) + (t % 32) // 2 + 16 * (t % 2)
                 for t in range(128))
_PERM64 = _PERM128[:64]


def _split_cols(W, perm=None):
    h0, h1 = W[:, :, 0:64], W[:, :, 64:128]
    if perm is not None:
        p = jnp.asarray(perm)
        h0, h1 = h0[:, :, p], h1[:, :, p]
    return jnp.stack([h0, h1], axis=1)


def _split_cols_b(b, perm=None):
    h0, h1 = b[:, None, 0:64], b[:, None, 64:128]
    if perm is not None:
        p = jnp.asarray(perm)
        h0, h1 = h0[:, :, p], h1[:, :, p]
    return jnp.stack([h0, h1], axis=1)


@jax.jit
def kernel(feature, edge_index, batch, emb_W, emb_b, A_W, A_b, B_W, B_b,
           D_W, D_b, E_W, E_b, bn_gamma, bn_beta, W1, b1, W2, b2, W3, b3):
    src = edge_index[0]
    dst = edge_index[1]
    pad = E2 - E
    zpad = jnp.zeros((pad,), jnp.int32)
    src_g = jnp.concatenate([src, zpad])
    dst_g = jnp.concatenate([dst, zpad])
    dst_s = jnp.concatenate([dst, jnp.full((pad,), N, jnp.int32)])
    idxD = dst_g
    idxEB = jnp.stack([src_g, src_g + N])

    nl2e = jnp.float32(-1.0)
    Dfull = jnp.broadcast_to(D_W[:, None] * nl2e, (L, 2, H, H))
    Dfull_b = jnp.broadcast_to(D_b[:, None, None] * nl2e, (L, 2, 1, H))
    Wc = jnp.concatenate(
        [Dfull, _split_cols(A_W), _split_cols(E_W) * nl2e,
         _split_cols(B_W)], axis=-1)
    bcat = jnp.concatenate(
        [Dfull_b, _split_cols_b(A_b), _split_cols_b(E_b) * nl2e,
         _split_cols_b(B_b)], axis=-1)

    W3p = jnp.pad(W3, ((0, 0), (0, H - NCLS)))
    b3p = jnp.pad(b3, (0, H - NCLS))
    batch2d = batch.reshape(N, 1)

    h = _embed(feature, emb_W, emb_b)
    h_in = h
    Ah2, Dh2, EB2 = _mm(h, Wc[0], bcat[0])
    for l in range(L):
        numden = _edge_pass(Dh2, EB2, dst_s, idxD, idxEB)
        hpre, stats = _update(Ah2, numden)
        if l < L - 1:
            h_in, Ah2, Dh2, EB2 = _norm_mm(
                hpre, stats, bn_gamma[l], bn_beta[l], h_in,
                Wc[l + 1], bcat[l + 1])
        else:
            y = _readout(hpre, stats, bn_gamma[l], bn_beta[l], h_in,
                         batch2d, W1, b1, W2, b2, W3p, b3p)
    return y[:, :NCLS]


# speedup vs baseline: 7.2630x; 1.0003x over previous
import jax
import jax.numpy as jnp
from jax import lax
from jax.experimental import pallas as pl
from jax.experimental.pallas import tpu as pltpu
from jax.experimental.pallas import tpu_sc as plsc

N = 10000
E = 320000
H = 128
L = 4
G = 64
NCLS = 10

NSC = 2
NTEC = 16
CH = 64
NCHUNK = 314
EPT = NCHUNK * CH
E2 = EPT * NTEC
NPAIR = NCHUNK // 2
RPT = 624

BLK = 1000
NB = N // BLK


def _embed_body(x_ref, w_ref, b_ref, o_ref):
    o_ref[...] = (
        jnp.dot(x_ref[...], w_ref[...], preferred_element_type=jnp.float32)
        + b_ref[...]
    )


def _embed(feature, emb_W, emb_b):
    return pl.pallas_call(
        _embed_body,
        grid=(NB,),
        in_specs=[
            pl.BlockSpec((BLK, H), lambda i: (i, 0)),
            pl.BlockSpec((H, H), lambda i: (0, 0)),
            pl.BlockSpec((1, H), lambda i: (0, 0)),
        ],
        out_specs=pl.BlockSpec((BLK, H), lambda i: (i, 0)),
        out_shape=jax.ShapeDtypeStruct((N, H), jnp.float32),
    )(feature, emb_W, emb_b.reshape(1, H))


def _mm_body(h_ref, w_ref, b_ref, a_ref, d_ref, eb_ref):
    out = (
        jnp.dot(h_ref[...], w_ref[0], preferred_element_type=jnp.float32)
        + b_ref[0]
    )
    d_ref[...] = out[:, 0:128]
    e16 = lax.bitcast_convert_type(
        out[:, 192:256].astype(jnp.bfloat16), jnp.uint16)
    b16 = lax.bitcast_convert_type(
        out[:, 256:320].astype(jnp.bfloat16), jnp.uint16)
    w = (b16.astype(jnp.uint32) << 16) | e16.astype(jnp.uint32)
    ebp = lax.bitcast_convert_type(w, jnp.float32)
    a_ref[...] = out[:, 128:192]
    eb_ref[...] = jnp.concatenate([ebp, ebp], axis=1)


def _mm(h, Wc, bc):
    return pl.pallas_call(
        _mm_body,
        grid=(NB, 2),
        in_specs=[
            pl.BlockSpec((BLK, H), lambda i, c: (i, 0)),
            pl.BlockSpec((1, H, 5 * 64), lambda i, c: (c, 0, 0)),
            pl.BlockSpec((1, 1, 5 * 64), lambda i, c: (c, 0, 0)),
        ],
        out_specs=[
            pl.BlockSpec((BLK, 64), lambda i, c: (c * NB + i, 0)),
            pl.BlockSpec((BLK, H), lambda i, c: (i, 0)),
            pl.BlockSpec((BLK, H), lambda i, c: (c * NB + i, 0)),
        ],
        out_shape=[
            jax.ShapeDtypeStruct((2 * N, 64), jnp.float32),
            jax.ShapeDtypeStruct((N, H), jnp.float32),
            jax.ShapeDtypeStruct((2 * N, H), jnp.float32),
        ],
    )(h, Wc, bc)


def _norm_mm_body(hp_ref, st_ref, g_ref, bt_ref, hin_ref, w_ref, b_ref,
                  h_ref, a_ref, d_ref, eb_ref):
    mean = st_ref[0] * (1.0 / N)
    var = st_ref[1] * (1.0 / N) - mean * mean
    rstd = lax.rsqrt(var + 1e-5)
    h = (
        jnp.maximum((hp_ref[...] - mean) * (rstd * g_ref[0]) + bt_ref[0], 0.0)
        + hin_ref[...]
    )
    h_ref[...] = h
    out = jnp.dot(h, w_ref[0], preferred_element_type=jnp.float32) + b_ref[0]
    d_ref[...] = out[:, 0:128]
    e16 = lax.bitcast_convert_type(
        out[:, 192:256].astype(jnp.bfloat16), jnp.uint16)
    b16 = lax.bitcast_convert_type(
        out[:, 256:320].astype(jnp.bfloat16), jnp.uint16)
    w = (b16.astype(jnp.uint32) << 16) | e16.astype(jnp.uint32)
    ebp = lax.bitcast_convert_type(w, jnp.float32)
    a_ref[...] = out[:, 128:192]
    eb_ref[...] = jnp.concatenate([ebp, ebp], axis=1)


def _norm_mm(hpre, stats, gamma, beta, h_in, Wc, bc):
    return pl.pallas_call(
        _norm_mm_body,
        grid=(NB, 2),
        in_specs=[
            pl.BlockSpec((BLK, H), lambda i, c: (i, 0)),
            pl.BlockSpec((2, H), lambda i, c: (0, 0)),
            pl.BlockSpec((1, H), lambda i, c: (0, 0)),
            pl.BlockSpec((1, H), lambda i, c: (0, 0)),
            pl.BlockSpec((BLK, H), lambda i, c: (i, 0)),
            pl.BlockSpec((1, H, 5 * 64), lambda i, c: (c, 0, 0)),
            pl.BlockSpec((1, 1, 5 * 64), lambda i, c: (c, 0, 0)),
        ],
        out_specs=[
            pl.BlockSpec((BLK, H), lambda i, c: (i, 0)),
            pl.BlockSpec((BLK, 64), lambda i, c: (c * NB + i, 0)),
            pl.BlockSpec((BLK, H), lambda i, c: (i, 0)),
            pl.BlockSpec((BLK, H), lambda i, c: (c * NB + i, 0)),
        ],
        out_shape=[
            jax.ShapeDtypeStruct((N, H), jnp.float32),
            jax.ShapeDtypeStruct((2 * N, 64), jnp.float32),
            jax.ShapeDtypeStruct((N, H), jnp.float32),
            jax.ShapeDtypeStruct((2 * N, H), jnp.float32),
        ],
    )(hpre, stats, gamma.reshape(1, H), beta.reshape(1, H), h_in, Wc, bc)


def _update_body(a0_ref, a1_ref, nd0_ref, nd1_ref, hp_ref, st_ref, acc_ref):
    i = pl.program_id(0)
    nd0 = nd0_ref[...]
    nd1 = nd1_ref[...]
    num = jnp.concatenate([nd0[:, 64:128], nd1[:, 64:128]], axis=1)
    den = jnp.concatenate([nd0[:, 0:64], nd1[:, 0:64]], axis=1)
    ah = jnp.concatenate([a0_ref[...], a1_ref[...]], axis=1)
    hp = ah + num / (den + 1e-6)
    hp_ref[...] = hp
    s1 = jnp.sum(hp, axis=0, keepdims=True)
    s2 = jnp.sum(hp * hp, axis=0, keepdims=True)
    s = jnp.concatenate([s1, s2], axis=0)

    @pl.when(i == 0)
    def _():
        acc_ref[...] = s

    @pl.when(i > 0)
    def _():
        acc_ref[...] += s

    st_ref[...] = acc_ref[...]


def _update(Ah2, numden):
    return pl.pallas_call(
        _update_body,
        grid=(NB,),
        in_specs=[
            pl.BlockSpec((BLK, 64), lambda i: (i, 0)),
            pl.BlockSpec((BLK, 64), lambda i: (NB + i, 0)),
            pl.BlockSpec((BLK, H), lambda i: (i, 0)),
            pl.BlockSpec((BLK, H), lambda i: (NB + i, 0)),
        ],
        out_specs=[
            pl.BlockSpec((BLK, H), lambda i: (i, 0)),
            pl.BlockSpec((2, H), lambda i: (0, 0)),
        ],
        out_shape=[
            jax.ShapeDtypeStruct((N, H), jnp.float32),
            jax.ShapeDtypeStruct((2, H), jnp.float32),
        ],
        scratch_shapes=[pltpu.VMEM((2, H), jnp.float32)],
    )(Ah2, Ah2, numden, numden)


def _read_body(hp_ref, st_ref, g_ref, bt_ref, hin_ref, b_ref,
               w1_ref, b1_ref, w2_ref, b2_ref, w3_ref, b3_ref,
               y_ref, hs_ref, cnt_ref):
    i = pl.program_id(0)
    mean = st_ref[0] * (1.0 / N)
    var = st_ref[1] * (1.0 / N) - mean * mean
    rstd = lax.rsqrt(var + 1e-5)
    h = (
        jnp.maximum((hp_ref[...] - mean) * (rstd * g_ref[0]) + bt_ref[0], 0.0)
        + hin_ref[...]
    )
    gid = lax.broadcasted_iota(jnp.int32, (BLK, G), 1)
    oh = (b_ref[...] == gid).astype(jnp.float32)
    dnums = (((0,), (0,)), ((), ()))
    hs = lax.dot_general(oh, h, dnums, preferred_element_type=jnp.float32)
    cn = lax.dot_general(oh, jnp.ones_like(h), dnums,
                         preferred_element_type=jnp.float32)

    @pl.when(i == 0)
    def _():
        hs_ref[...] = hs
        cnt_ref[...] = cn

    @pl.when(i > 0)
    def _():
        hs_ref[...] += hs
        cnt_ref[...] += cn

    @pl.when(i == NB - 1)
    def _():
        hg = hs_ref[...] / jnp.maximum(cnt_ref[...], 1.0)
        y1 = jnp.maximum(
            jnp.dot(hg, w1_ref[...], preferred_element_type=jnp.float32)
            + b1_ref[0], 0.0)
        y2 = jnp.maximum(
            jnp.dot(y1, w2_ref[...], preferred_element_type=jnp.float32)
            + b2_ref[0], 0.0)
        y_ref[...] = (
            jnp.dot(y2, w3_ref[...], preferred_element_type=jnp.float32)
            + b3_ref[0]
        )


def _readout(hpre, stats, gamma, beta, h_in, batch2d, W1, b1, W2, b2, W3p, b3p):
    return pl.pallas_call(
        _read_body,
        grid=(NB,),
        in_specs=[
            pl.BlockSpec((BLK, H), lambda i: (i, 0)),
            pl.BlockSpec((2, H), lambda i: (0, 0)),
            pl.BlockSpec((1, H), lambda i: (0, 0)),
            pl.BlockSpec((1, H), lambda i: (0, 0)),
            pl.BlockSpec((BLK, H), lambda i: (i, 0)),
            pl.BlockSpec((BLK, 1), lambda i: (i, 0)),
            pl.BlockSpec((H, 64), lambda i: (0, 0)),
            pl.BlockSpec((1, 64), lambda i: (0, 0)),
            pl.BlockSpec((64, 32), lambda i: (0, 0)),
            pl.BlockSpec((1, 32), lambda i: (0, 0)),
            pl.BlockSpec((32, H), lambda i: (0, 0)),
            pl.BlockSpec((1, H), lambda i: (0, 0)),
        ],
        out_specs=pl.BlockSpec((G, H), lambda i: (0, 0)),
        out_shape=jax.ShapeDtypeStruct((G, H), jnp.float32),
        scratch_shapes=[
            pltpu.VMEM((G, H), jnp.float32),
            pltpu.VMEM((G, H), jnp.float32),
        ],
    )(hpre, stats, gamma.reshape(1, H), beta.reshape(1, H), h_in, batch2d,
      W1, b1.reshape(1, 64), W2, b2.reshape(1, 32), W3p, b3p.reshape(1, H))


def _edge_body(dh_hbm, eb2_hbm, dsts_hbm, idxd_hbm, idxeb_hbm, out_hbm,
               dst_v0, dst_v1, idd_v0, idd_v1, ideb_v0, ideb_v1,
               dsc_v0, dsc_v1, dr0, dr1, eb0, eb1, ms0, ms1, acc,
               semi0, semi1, semg0, semg1, semh0, semh1, sems0, sems1):
    c = lax.axis_index("c")
    s = lax.axis_index("s")
    dst_v = (dst_v0, dst_v1)
    idd_v = (idd_v0, idd_v1)
    ideb_v = (ideb_v0, ideb_v1)
    dsc_v = (dsc_v0, dsc_v1)
    drows = (dr0, dr1)
    ebrows = (eb0, eb1)
    msgsig = (ms0, ms1)
    semi = (semi0, semi1)
    semg = (semg0, semg1)
    semh = (semh0, semh1)
    sems = (sems0, sems1)

    def _zb16(j, carry):
        r = j // 8
        q = j % 8
        ms0[r, pl.ds(q * 16, 16)] = jnp.zeros((16,), jnp.float32)
        return carry

    lax.fori_loop(0, CH * 8, _zb16, 0)
    row0 = s * RPT
    for t in range(9):
        pltpu.sync_copy(ms0, acc.at[pl.ds(row0 + t * CH, CH)])
    pltpu.sync_copy(ms0.at[pl.ds(0, RPT - 9 * CH)],
                    acc.at[pl.ds(row0 + 9 * CH, RPT - 9 * CH)])

    @pl.when(s == NTEC - 1)
    def _():
        pltpu.sync_copy(ms0.at[pl.ds(0, 24)], acc.at[pl.ds(NTEC * RPT, 24)])

    plsc.subcore_barrier()

    base0 = s * EPT
    coff = c * 64

    def _load_idx_sync(b, base):
        pltpu.sync_copy(dsts_hbm.at[pl.ds(base, CH)], dst_v[b])
        pltpu.sync_copy(idxd_hbm.at[pl.ds(base, CH)], idd_v[b])
        pltpu.sync_copy(idxeb_hbm.at[c, pl.ds(base, CH)], ideb_v[b])

    def _issue_idx(b, base):
        pltpu.async_copy(dsts_hbm.at[pl.ds(base, CH)], dst_v[b], semi[b])
        pltpu.async_copy(idxd_hbm.at[pl.ds(base, CH)], idd_v[b], semi[b])
        pltpu.async_copy(idxeb_hbm.at[c, pl.ds(base, CH)], ideb_v[b], semi[b])

    def _wait_idx(b, base):
        pltpu.make_async_copy(dsts_hbm.at[pl.ds(base, CH)], dst_v[b],
                              semi[b]).wait()
        pltpu.make_async_copy(idxd_hbm.at[pl.ds(base, CH)], idd_v[b],
                              semi[b]).wait()
        pltpu.make_async_copy(idxeb_hbm.at[c, pl.ds(base, CH)], ideb_v[b],
                              semi[b]).wait()

    def _issue_gathers(b):
        pltpu.async_copy(dh_hbm.at[idd_v[b]], drows[b], semg[b])
        pltpu.async_copy(eb2_hbm.at[ideb_v[b]], ebrows[b], semh[b])

    def _wait_gathers(b):
        pltpu.make_async_copy(dh_hbm.at[idd_v[b]], drows[b], semg[b]).wait()
        pltpu.make_async_copy(eb2_hbm.at[ideb_v[b]], ebrows[b],
                              semh[b]).wait()

    def _compute(b):
        dr = drows[b]
        eb = ebrows[b]
        ms = msgsig[b]

        def _quad(rq, rc):
            r0 = 4 * rq
            vals = []
            for r_ in range(4):
                r = r0 + r_
                for q in range(4):
                    d = dr[r, pl.ds(coff + q * 16, 16)]
                    w = plsc.bitcast(eb[r, pl.ds(q * 16, 16)], jnp.int32)
                    e = plsc.bitcast(w << 16, jnp.float32)
                    bb = plsc.bitcast(w & jnp.int32(-65536), jnp.float32)
                    vals.append((d + e, bb))
            es = [jnp.exp(x) for x, _ in vals]
            sigs = [1.0 / (1.0 + t_) for t_ in es]
            k = 0
            for r_ in range(4):
                r = r0 + r_
                for q in range(4):
                    ms[r, pl.ds(q * 16, 16)] = sigs[k]
                    ms[r, pl.ds(64 + q * 16, 16)] = sigs[k] * vals[k][1]
                    k += 1
            return rc

        lax.fori_loop(0, CH // 4, _quad, 0)

    for b in range(2):
        _load_idx_sync(b, base0 + b * CH)
        _issue_gathers(b)

    def _pair(k2, carry):
        for b in range(2):
            k = 2 * k2 + b
            base_next = base0 + (k + 2) * CH
            _wait_gathers(b)

            @pl.when(k >= 2)
            def _():
                pltpu.make_async_copy(msgsig[b], acc.at[dsc_v[b]],
                                      sems[b]).wait()

            for q in range(4):
                dsc_v[b][pl.ds(q * 16, 16)] = dst_v[b][pl.ds(q * 16, 16)]

            @pl.when(k < NCHUNK - 2)
            def _():
                _issue_idx(b, base_next)

            _compute(b)
            pltpu.async_copy(msgsig[b], acc.at[dsc_v[b]], sems[b], add=True)

            @pl.when(k < NCHUNK - 2)
            def _():
                _wait_idx(b, base_next)
                _issue_gathers(b)

        return carry

    lax.fori_loop(0, NPAIR, _pair, 0)

    for b in range(2):
        pltpu.make_async_copy(msgsig[b], acc.at[dsc_v[b]], sems[b]).wait()

    plsc.subcore_barrier()
    pltpu.sync_copy(acc.at[pl.ds(row0, RPT)],
                    out_hbm.at[pl.ds(c * N + row0, RPT)])

    @pl.when(s == NTEC - 1)
    def _():
        pltpu.sync_copy(acc.at[pl.ds(NTEC * RPT, 16)],
                        out_hbm.at[pl.ds(c * N + NTEC * RPT, 16)])


def _edge_pass(Dh, EB2, dst_s, idxD, idxEB):
    mesh = plsc.VectorSubcoreMesh(core_axis_name="c", subcore_axis_name="s",
                                  num_cores=NSC, num_subcores=NTEC)
    f = pl.kernel(
        _edge_body,
        out_type=jax.ShapeDtypeStruct((2 * N, H), jnp.float32),
        mesh=mesh,
        compiler_params=pltpu.CompilerParams(needs_layout_passes=False),
        scratch_types=(
            [pltpu.VMEM((CH,), jnp.int32) for _ in range(8)]
            + [pltpu.VMEM((CH, H), jnp.float32) for _ in range(6)]
            + [pltpu.VMEM_SHARED((N + 8, H), jnp.float32)]
            + [pltpu.SemaphoreType.DMA for _ in range(8)]
        ),
    )
    return f(Dh, EB2, dst_s, idxD, idxEB)


def _split_cols(W):
    return jnp.stack([W[:, :, 0:64], W[:, :, 64:128]], axis=1)


def _split_cols_b(b):
    return jnp.stack([b[:, None, 0:64], b[:, None, 64:128]], axis=1)


@jax.jit
def kernel(feature, edge_index, batch, emb_W, emb_b, A_W, A_b, B_W, B_b,
           D_W, D_b, E_W, E_b, bn_gamma, bn_beta, W1, b1, W2, b2, W3, b3):
    src = edge_index[0]
    dst = edge_index[1]
    pad = E2 - E
    zpad = jnp.zeros((pad,), jnp.int32)
    src_g = jnp.concatenate([src, zpad])
    dst_g = jnp.concatenate([dst, zpad])
    dst_s = jnp.concatenate([dst, jnp.full((pad,), N, jnp.int32)])
    idxD = dst_g
    idxEB = jnp.stack([src_g, src_g + N])

    nl2e = jnp.float32(-1.0)
    Dfull = jnp.broadcast_to(D_W[:, None] * nl2e, (L, 2, H, H))
    Dfull_b = jnp.broadcast_to(D_b[:, None, None] * nl2e, (L, 2, 1, H))
    Wc = jnp.concatenate(
        [Dfull, _split_cols(A_W), _split_cols(E_W) * nl2e,
         _split_cols(B_W)], axis=-1)
    bcat = jnp.concatenate(
        [Dfull_b, _split_cols_b(A_b), _split_cols_b(E_b) * nl2e,
         _split_cols_b(B_b)], axis=-1)

    W3p = jnp.pad(W3, ((0, 0), (0, H - NCLS)))
    b3p = jnp.pad(b3, (0, H - NCLS))
    batch2d = batch.reshape(N, 1)

    h = _embed(feature, emb_W, emb_b)
    h_in = h
    Ah2, Dh2, EB2 = _mm(h, Wc[0], bcat[0])
    for l in range(L):
        numden = _edge_pass(Dh2, EB2, dst_s, idxD, idxEB)
        hpre, stats = _update(Ah2, numden)
        if l < L - 1:
            h_in, Ah2, Dh2, EB2 = _norm_mm(
                hpre, stats, bn_gamma[l], bn_beta[l], h_in,
                Wc[l + 1], bcat[l + 1])
        else:
            y = _readout(hpre, stats, bn_gamma[l], bn_beta[l], h_in,
                         batch2d, W1, b1, W2, b2, W3p, b3p)
    return y[:, :NCLS]
